# Initial kernel scaffold; baseline (speedup 1.0000x reference)
import jax
import jax.numpy as jnp
from jax.experimental import pallas as pl


def kernel(latent, genes_oi, cells_oi, cut_coordinates, cut_local_cellxgene_ix, cut_local_gene_ix, local_cellxgene_ix, n_cells, n_genes, logit_weight, rho_weight, bin_logit_baseline, rho_bias, libsize):
    raise NotImplementedError("write your pallas kernel here")


# speedup vs baseline: 2.3240x; 2.3240x over previous
import functools

import jax
import jax.numpy as jnp
from jax import lax
from jax.experimental import pallas as pl
from jax.experimental.pallas import tpu as pltpu
from jax.experimental.pallas import tpu_sc as plsc

B = 256
G = 500
L = 32
C = 128
NGT = 5000
NC = 200000
NF = 400000
NW = 32
CH = 128
NC_CHUNKS = 1568
NCPAD = NC_CHUNKS * CH
NF_CHUNKS = 3136
NFPAD = NF_CHUNKS * CH
H = 131072
BG = B * G


def _k1_body(genes_ref, latent_ref, lw_ref, md_ref):
    del genes_ref
    latent = latent_ref[...]
    lw = lw_ref[...].reshape(L, C)
    md_ref[...] = jnp.dot(latent, lw, preferred_element_type=jnp.float32,
                          precision=lax.Precision.HIGHEST)


def _k1(latent, genes_oi, logit_weight):
    grid_spec = pltpu.PrefetchScalarGridSpec(
        num_scalar_prefetch=1,
        grid=(G,),
        in_specs=[
            pl.BlockSpec((B, L), lambda g, gref: (0, 0)),
            pl.BlockSpec((1, L, C), lambda g, gref: (gref[g], 0, 0)),
        ],
        out_specs=[
            pl.BlockSpec((B, C), lambda g, gref: (g, 0)),
        ],
    )
    return pl.pallas_call(
        _k1_body,
        grid_spec=grid_spec,
        out_shape=[
            jax.ShapeDtypeStruct((G * B, C), jnp.float32),
        ],
    )(genes_oi, latent, logit_weight)[0]


def _k2_body(md_hbm, bl_hbm, genes_hbm, cxg_hbm, gix_hbm, coord_hbm,
             frag_hbm, rb_hbm, ls_hbm, cells_hbm, rw_hbm,
             pm_hbm, s_hbm, hist_hbm, rboi_hbm, lsoi_hbm, rwoi_hbm,
             genes_v, idx_v, gix_v, gg_v, coord_v, bins_v,
             mdrows_v, blrows_v, pm_v, s_v,
             frag_v, ones_v, zeros_v,
             rb_v, ls_v, cells_v, rboi_v, lsoi_v, gidx_v, rwoi_v,
             hist_sh, sem1, sem2):
    c = lax.axis_index("c")
    s = lax.axis_index("s")
    wid = s * 2 + c

    pltpu.sync_copy(genes_hbm, genes_v)

    @pl.when(s == 0)
    def _zero_hist():
        def zv(i, carry):
            zeros_v[pl.ds(i * 16, 16)] = jnp.zeros((16,), jnp.int32)
            return carry
        lax.fori_loop(0, 128, zv, 0)

        def zh(k, carry):
            pltpu.sync_copy(zeros_v, hist_sh.at[pl.ds(k * 2048, 2048)])
            return carry
        lax.fori_loop(0, H // 2048, zh, 0)

    for k in range(8):
        ones_v[pl.ds(k * 16, 16)] = jnp.full((16,), 1, jnp.int32)

    plsc.subcore_barrier()

    def frag_step(j, carry):
        toff = pl.multiple_of((j * NW + wid) * CH, CH)
        pltpu.sync_copy(frag_hbm.at[pl.ds(toff, CH)], frag_v)
        pltpu.sync_copy(ones_v, hist_sh.at[frag_v], add=True)
        return carry
    lax.fori_loop(0, NF_CHUNKS // NW, frag_step, 0)

    def cut_step(j, carry):
        toff = pl.multiple_of((j * NW + wid) * CH, CH)
        pltpu.sync_copy(cxg_hbm.at[pl.ds(toff, CH)], idx_v)
        pltpu.sync_copy(gix_hbm.at[pl.ds(toff, CH)], gix_v)
        pltpu.sync_copy(coord_hbm.at[pl.ds(toff, CH)], coord_v)
        for k in range(CH // 16):
            sl = pl.ds(k * 16, 16)
            ix = idx_v[sl]
            idx_v[sl] = lax.rem(ix, jnp.int32(G)) * B + lax.div(ix, jnp.int32(G))
            gg_v[sl] = plsc.load_gather(genes_v, [gix_v[sl]])
            b = (coord_v[sl] * jnp.float32(C)).astype(jnp.int32)
            bins_v[sl] = jnp.clip(b, 0, C - 1)
        cp1 = pltpu.async_copy(md_hbm.at[idx_v], mdrows_v, sem1)
        cp2 = pltpu.async_copy(bl_hbm.at[gg_v], blrows_v, sem2)
        cp1.wait()
        cp2.wait()

        lane0 = lax.iota(jnp.int32, 16) == 0

        def cut_body(i, carry2):
            r = [mdrows_v[i, pl.ds(k * 16, 16)] + blrows_v[i, pl.ds(k * 16, 16)]
                 for k in range(C // 16)]
            m01 = jnp.maximum(jnp.maximum(r[0], r[1]), jnp.maximum(r[2], r[3]))
            m23 = jnp.maximum(jnp.maximum(r[4], r[5]), jnp.maximum(r[6], r[7]))
            m = jnp.max(jnp.maximum(m01, m23))
            acc = jnp.zeros((16,), jnp.float32)
            for k in range(C // 16):
                acc = acc + jnp.exp(r[k] - m)
            sval = jnp.sum(acc)
            ii = jnp.full((16,), i, jnp.int32)
            colb = plsc.load_gather(bins_v, [ii])
            p16 = (plsc.load_gather(mdrows_v, [ii, colb])
                   + plsc.load_gather(blrows_v, [ii, colb]))
            plsc.store_scatter(pm_v, [ii], p16 - m, mask=lane0)
            plsc.store_scatter(s_v, [ii], sval + jnp.zeros((16,), jnp.float32),
                               mask=lane0)
            return carry2
        lax.fori_loop(0, CH, cut_body, 0)

        pltpu.sync_copy(pm_v, pm_hbm.at[pl.ds(toff, CH)])
        pltpu.sync_copy(s_v, s_hbm.at[pl.ds(toff, CH)])
        return carry
    lax.fori_loop(0, NC_CHUNKS // NW, cut_step, 0)

    @pl.when(jnp.logical_and(c == 0, s == 0))
    def _small_gathers():
        pltpu.sync_copy(rb_hbm, rb_v)
        pltpu.sync_copy(ls_hbm, ls_v)
        pltpu.sync_copy(cells_hbm, cells_v)
        for k in range(512 // 16):
            sl = pl.ds(k * 16, 16)
            rboi_v[sl] = plsc.load_gather(rb_v, [genes_v[sl]])
        for k in range(256 // 16):
            sl = pl.ds(k * 16, 16)
            lsoi_v[sl] = plsc.load_gather(ls_v, [cells_v[sl]])
        pltpu.sync_copy(rboi_v, rboi_hbm)
        pltpu.sync_copy(lsoi_v, lsoi_hbm)
        for k in range(512 // CH):
            pltpu.sync_copy(genes_hbm.at[pl.ds(k * CH, CH)], gidx_v)
            pltpu.async_copy(rw_hbm.at[gidx_v], rwoi_v, sem1).wait()
            pltpu.sync_copy(rwoi_v, rwoi_hbm.at[pl.ds(k * CH, CH)])

    plsc.subcore_barrier()

    @pl.when(s == 0)
    def _export_hist():
        pltpu.sync_copy(hist_sh, hist_hbm.at[c])


def _k2(md_flat, bl, genes_pad, cxg_pad, gix_pad, coord_pad, frag_pad,
        rb_pad, ls, cells, rw):
    kfn = functools.partial(
        pl.kernel,
        out_type=[
            jax.ShapeDtypeStruct((NCPAD,), jnp.float32),
            jax.ShapeDtypeStruct((NCPAD,), jnp.float32),
            jax.ShapeDtypeStruct((2, H), jnp.int32),
            jax.ShapeDtypeStruct((512,), jnp.float32),
            jax.ShapeDtypeStruct((256,), jnp.float32),
            jax.ShapeDtypeStruct((512, C), jnp.float32),
        ],
        mesh=plsc.VectorSubcoreMesh(core_axis_name="c", subcore_axis_name="s"),
        compiler_params=pltpu.CompilerParams(needs_layout_passes=False),
        scratch_types=[
            pltpu.VMEM((512,), jnp.int32),
            pltpu.VMEM((CH,), jnp.int32),
            pltpu.VMEM((CH,), jnp.int32),
            pltpu.VMEM((CH,), jnp.int32),
            pltpu.VMEM((CH,), jnp.float32),
            pltpu.VMEM((CH,), jnp.int32),
            pltpu.VMEM((CH, C), jnp.float32),
            pltpu.VMEM((CH, C), jnp.float32),
            pltpu.VMEM((CH,), jnp.float32),
            pltpu.VMEM((CH,), jnp.float32),
            pltpu.VMEM((CH,), jnp.int32),
            pltpu.VMEM((CH,), jnp.int32),
            pltpu.VMEM((2048,), jnp.int32),
            pltpu.VMEM((5120,), jnp.float32),
            pltpu.VMEM((10000,), jnp.float32),
            pltpu.VMEM((256,), jnp.int32),
            pltpu.VMEM((512,), jnp.float32),
            pltpu.VMEM((256,), jnp.float32),
            pltpu.VMEM((CH,), jnp.int32),
            pltpu.VMEM((CH, C), jnp.float32),
            pltpu.VMEM_SHARED((H,), jnp.int32),
            pltpu.SemaphoreType.DMA,
            pltpu.SemaphoreType.DMA,
        ],
    )
    return kfn(_k2_body)(md_flat, bl, genes_pad, cxg_pad, gix_pad, coord_pad,
                         frag_pad, rb_pad, ls, cells, rw)


def _k3_body(pm_ref, s_ref, h0_ref, h1_ref, latent_ref, rw_ref, rb_ref,
             ls_ref, out_ref):
    pm = pm_ref[...]
    sv = s_ref[...]
    r0 = lax.broadcasted_iota(jnp.int32, pm.shape, 0)
    c0 = lax.broadcasted_iota(jnp.int32, pm.shape, 1)
    maskc = (r0 * 128 + c0) < NC
    mix = jnp.sum(jnp.where(maskc, pm - jnp.log(jnp.where(maskc, sv, 1.0)),
                            0.0))
    mix = mix + jnp.float32(NC) * jnp.log(jnp.float32(C))

    fc = (h0_ref[...] + h1_ref[...]).astype(jnp.float32)
    rho = lax.dot_general(latent_ref[...], rw_ref[...],
                          (((1,), (1,)), ((), ())),
                          preferred_element_type=jnp.float32,
                          precision=lax.Precision.HIGHEST)
    fe = rb_ref[...] * jnp.exp(rho) * ls_ref[...]
    x = fc + 1.0
    z = x + 7.0
    prod = (x * (x + 1.0) * (x + 2.0) * (x + 3.0) * (x + 4.0) * (x + 5.0)
            * (x + 6.0))
    zi = 1.0 / z
    zi2 = zi * zi
    lg = ((z - 0.5) * jnp.log(z) - z + jnp.float32(0.9189385332046727)
          + zi * (jnp.float32(1.0 / 12.0)
                  - zi2 * (jnp.float32(1.0 / 360.0)
                           - zi2 * jnp.float32(1.0 / 1260.0)))
          - jnp.log(prod))
    lfc = fc * jnp.log(fe) - fe - lg
    out_ref[0, 0] = -(mix + jnp.sum(lfc))


def _k3(pm2, s2, h0, h1, latent, rwoi, rb_row, ls_col):
    return pl.pallas_call(
        _k3_body,
        out_shape=jax.ShapeDtypeStruct((1, 1), jnp.float32),
        out_specs=pl.BlockSpec(memory_space=pltpu.SMEM),
    )(pm2, s2, h0, h1, latent, rwoi, rb_row, ls_col)


def kernel(latent, genes_oi, cells_oi, cut_coordinates, cut_local_cellxgene_ix,
           cut_local_gene_ix, local_cellxgene_ix, n_cells, n_genes,
           logit_weight, rho_weight, bin_logit_baseline, rho_bias, libsize):
    genes_oi = genes_oi.astype(jnp.int32)
    cells_oi = cells_oi.astype(jnp.int32)
    cxg = cut_local_cellxgene_ix.astype(jnp.int32)
    gix = cut_local_gene_ix.astype(jnp.int32)
    frag = local_cellxgene_ix.astype(jnp.int32)

    md_flat = _k1(latent, genes_oi, logit_weight)

    genes_pad = jnp.pad(genes_oi, (0, 512 - G))
    cxg_pad = jnp.pad(cxg, (0, NCPAD - NC))
    gix_pad = jnp.pad(gix, (0, NCPAD - NC))
    coord_pad = jnp.pad(cut_coordinates, (0, NCPAD - NC))
    frag_pad = jnp.pad(frag, (0, NFPAD - NF), constant_values=BG)
    rb_pad = jnp.pad(rho_bias, (0, 5120 - NGT))
    rw_pad = jnp.pad(rho_weight, ((0, 0), (0, C - L)))

    pm, sv, hist, rboi, lsoi, rwoi = _k2(md_flat, bin_logit_baseline,
                                         genes_pad, cxg_pad, gix_pad,
                                         coord_pad, frag_pad, rb_pad,
                                         libsize, cells_oi, rw_pad)

    pm2 = pm.reshape(NCPAD // 128, 128)
    s2 = sv.reshape(NCPAD // 128, 128)
    h0 = hist[0, :BG].reshape(B, G)
    h1 = hist[1, :BG].reshape(B, G)
    rb_row = rboi[:G].reshape(1, G)
    ls_col = lsoi.reshape(B, 1)

    out = _k3(pm2, s2, h0, h1, latent, rwoi[:G, :L], rb_row, ls_col)
    scale = (jnp.asarray(n_cells, jnp.float32) * jnp.asarray(n_genes, jnp.float32)
             / jnp.float32(BG))
    return out[0, 0] * scale


# speedup vs baseline: 2.9044x; 1.2498x over previous
import functools

import jax
import jax.numpy as jnp
from jax import lax
from jax.experimental import pallas as pl
from jax.experimental.pallas import tpu as pltpu
from jax.experimental.pallas import tpu_sc as plsc

B = 256
G = 500
L = 32
C = 128
NGT = 5000
NC = 200000
NF = 400000
NW = 32
CH = 128
NC_CHUNKS = 1568
NCPAD = NC_CHUNKS * CH
NF_CHUNKS = 3136
NFPAD = NF_CHUNKS * CH
H = 131072
BG = B * G


def _k1_body(genes_ref, latent_ref, lw_ref, md_ref):
    del genes_ref
    latent = latent_ref[...]
    lw = lw_ref[...].reshape(L, C)
    md_ref[...] = jnp.dot(latent, lw, preferred_element_type=jnp.float32,
                          precision=lax.Precision.HIGHEST)


def _k1(latent, genes_oi, logit_weight):
    grid_spec = pltpu.PrefetchScalarGridSpec(
        num_scalar_prefetch=1,
        grid=(G,),
        in_specs=[
            pl.BlockSpec((B, L), lambda g, gref: (0, 0)),
            pl.BlockSpec((1, L, C), lambda g, gref: (gref[g], 0, 0)),
        ],
        out_specs=[
            pl.BlockSpec((B, C), lambda g, gref: (g, 0)),
        ],
    )
    return pl.pallas_call(
        _k1_body,
        grid_spec=grid_spec,
        out_shape=[
            jax.ShapeDtypeStruct((G * B, C), jnp.float32),
        ],
    )(genes_oi, latent, logit_weight)[0]


def _k2_body(md_hbm, bl_hbm, genes_hbm, cxg_hbm, gix_hbm, coord_hbm,
             frag_hbm, rb_hbm, ls_hbm, cells_hbm, rw_hbm,
             pm_hbm, s_hbm, hist_hbm, rboi_hbm, lsoi_hbm, rwoi_hbm,
             genes_v, gix_v, coord_v, pm_v, s_v,
             idx_a, gg_a, bins_a, md_a, bl_a,
             idx_b, gg_b, bins_b, md_b, bl_b,
             frag_v, ones_v, zeros_v,
             rb_v, ls_v, cells_v, rboi_v, lsoi_v, gidx_v, rwoi_v,
             hist_sh, sem_md_a, sem_bl_a, sem_md_b, sem_bl_b, sem1):
    c = lax.axis_index("c")
    s = lax.axis_index("s")
    wid = s * 2 + c

    pltpu.sync_copy(genes_hbm, genes_v)

    @pl.when(s == 0)
    def _zero_hist():
        def zv(i, carry):
            zeros_v[pl.ds(i * 16, 16)] = jnp.zeros((16,), jnp.int32)
            return carry
        lax.fori_loop(0, 128, zv, 0)

        def zh(k, carry):
            pltpu.sync_copy(zeros_v, hist_sh.at[pl.ds(k * 2048, 2048)])
            return carry
        lax.fori_loop(0, H // 2048, zh, 0)

    for k in range(8):
        ones_v[pl.ds(k * 16, 16)] = jnp.full((16,), 1, jnp.int32)

    plsc.subcore_barrier()

    def frag_step(j, carry):
        toff = pl.multiple_of((j * NW + wid) * CH, CH)
        pltpu.sync_copy(frag_hbm.at[pl.ds(toff, CH)], frag_v)
        pltpu.sync_copy(ones_v, hist_sh.at[frag_v], add=True)
        return carry
    lax.fori_loop(0, NF_CHUNKS // NW, frag_step, 0)

    NJ = NC_CHUNKS // NW
    lane = lax.iota(jnp.int32, 16)
    lane0 = lane == 0

    def fire(t, idxb, ggb, binsb, mdb, blb, sem_md, sem_bl):
        toff = pl.multiple_of(t * CH, CH)
        pltpu.sync_copy(cxg_hbm.at[pl.ds(toff, CH)], idxb)
        pltpu.sync_copy(gix_hbm.at[pl.ds(toff, CH)], gix_v)
        pltpu.sync_copy(coord_hbm.at[pl.ds(toff, CH)], coord_v)
        for k in range(CH // 16):
            sl = pl.ds(k * 16, 16)
            ix = idxb[sl]
            idxb[sl] = lax.rem(ix, jnp.int32(G)) * B + lax.div(ix, jnp.int32(G))
            ggb[sl] = plsc.load_gather(genes_v, [gix_v[sl]])
            b = (coord_v[sl] * jnp.float32(C)).astype(jnp.int32)
            binsb[sl] = jnp.clip(b, 0, C - 1)
        pltpu.async_copy(md_hbm.at[idxb], mdb, sem_md)
        pltpu.async_copy(bl_hbm.at[ggb], blb, sem_bl)

    def compute(t, idxb, ggb, binsb, mdb, blb, sem_md, sem_bl):
        pltpu.make_async_copy(md_hbm.at[idxb], mdb, sem_md).wait()
        pltpu.make_async_copy(bl_hbm.at[ggb], blb, sem_bl).wait()
        toff = pl.multiple_of(t * CH, CH)
        for k in range(CH // 16):
            sl = pl.ds(k * 16, 16)
            rows = lane + jnp.int32(k * 16)
            cols = binsb[sl]
            pm_v[sl] = (plsc.load_gather(mdb, [rows, cols])
                        + plsc.load_gather(blb, [rows, cols]))

        def cut_body(i, carry2):
            acc = jnp.zeros((16,), jnp.float32)
            for k in range(C // 16):
                sl = pl.ds(k * 16, 16)
                acc = acc + jnp.exp(mdb[i, sl] + blb[i, sl])
            sval = jnp.sum(acc)
            ii = jnp.full((16,), i, jnp.int32)
            plsc.store_scatter(s_v, [ii], sval + jnp.zeros((16,), jnp.float32),
                               mask=lane0)
            return carry2
        lax.fori_loop(0, CH, cut_body, 0)

        pltpu.sync_copy(pm_v, pm_hbm.at[pl.ds(toff, CH)])
        pltpu.sync_copy(s_v, s_hbm.at[pl.ds(toff, CH)])

    bufs_a = (idx_a, gg_a, bins_a, md_a, bl_a, sem_md_a, sem_bl_a)
    bufs_b = (idx_b, gg_b, bins_b, md_b, bl_b, sem_md_b, sem_bl_b)
    fire(wid, *bufs_a)

    def cut_step(j, carry):
        tcur = j * NW + wid
        even = lax.rem(j, 2) == 0

        @pl.when(jnp.logical_and(even, j < NJ - 1))
        def _fb():
            fire(tcur + NW, *bufs_b)

        @pl.when(jnp.logical_and(jnp.logical_not(even), j < NJ - 1))
        def _fa():
            fire(tcur + NW, *bufs_a)

        @pl.when(even)
        def _ca():
            compute(tcur, *bufs_a)

        @pl.when(jnp.logical_not(even))
        def _cb():
            compute(tcur, *bufs_b)
        return carry
    lax.fori_loop(0, NJ, cut_step, 0)

    @pl.when(jnp.logical_and(c == 0, s == 0))
    def _small_gathers():
        pltpu.sync_copy(rb_hbm, rb_v)
        pltpu.sync_copy(ls_hbm, ls_v)
        pltpu.sync_copy(cells_hbm, cells_v)
        for k in range(512 // 16):
            sl = pl.ds(k * 16, 16)
            rboi_v[sl] = plsc.load_gather(rb_v, [genes_v[sl]])
        for k in range(256 // 16):
            sl = pl.ds(k * 16, 16)
            lsoi_v[sl] = plsc.load_gather(ls_v, [cells_v[sl]])
        pltpu.sync_copy(rboi_v, rboi_hbm)
        pltpu.sync_copy(lsoi_v, lsoi_hbm)
        for k in range(512 // CH):
            pltpu.sync_copy(genes_hbm.at[pl.ds(k * CH, CH)], gidx_v)
            pltpu.async_copy(rw_hbm.at[gidx_v], rwoi_v, sem1).wait()
            pltpu.sync_copy(rwoi_v, rwoi_hbm.at[pl.ds(k * CH, CH)])

    plsc.subcore_barrier()

    @pl.when(s == 0)
    def _export_hist():
        pltpu.sync_copy(hist_sh, hist_hbm.at[c])


def _k2(md_flat, bl, genes_pad, cxg_pad, gix_pad, coord_pad, frag_pad,
        rb_pad, ls, cells, rw):
    kfn = functools.partial(
        pl.kernel,
        out_type=[
            jax.ShapeDtypeStruct((NCPAD,), jnp.float32),
            jax.ShapeDtypeStruct((NCPAD,), jnp.float32),
            jax.ShapeDtypeStruct((2, H), jnp.int32),
            jax.ShapeDtypeStruct((512,), jnp.float32),
            jax.ShapeDtypeStruct((256,), jnp.float32),
            jax.ShapeDtypeStruct((512, C), jnp.float32),
        ],
        mesh=plsc.VectorSubcoreMesh(core_axis_name="c", subcore_axis_name="s"),
        compiler_params=pltpu.CompilerParams(needs_layout_passes=False),
        scratch_types=[
            pltpu.VMEM((512,), jnp.int32),
            pltpu.VMEM((CH,), jnp.int32),
            pltpu.VMEM((CH,), jnp.float32),
            pltpu.VMEM((CH,), jnp.float32),
            pltpu.VMEM((CH,), jnp.float32),
            pltpu.VMEM((CH,), jnp.int32),
            pltpu.VMEM((CH,), jnp.int32),
            pltpu.VMEM((CH,), jnp.int32),
            pltpu.VMEM((CH, C), jnp.float32),
            pltpu.VMEM((CH, C), jnp.float32),
            pltpu.VMEM((CH,), jnp.int32),
            pltpu.VMEM((CH,), jnp.int32),
            pltpu.VMEM((CH,), jnp.int32),
            pltpu.VMEM((CH, C), jnp.float32),
            pltpu.VMEM((CH, C), jnp.float32),
            pltpu.VMEM((CH,), jnp.int32),
            pltpu.VMEM((CH,), jnp.int32),
            pltpu.VMEM((2048,), jnp.int32),
            pltpu.VMEM((5120,), jnp.float32),
            pltpu.VMEM((10000,), jnp.float32),
            pltpu.VMEM((256,), jnp.int32),
            pltpu.VMEM((512,), jnp.float32),
            pltpu.VMEM((256,), jnp.float32),
            pltpu.VMEM((CH,), jnp.int32),
            pltpu.VMEM((CH, C), jnp.float32),
            pltpu.VMEM_SHARED((H,), jnp.int32),
            pltpu.SemaphoreType.DMA,
            pltpu.SemaphoreType.DMA,
            pltpu.SemaphoreType.DMA,
            pltpu.SemaphoreType.DMA,
            pltpu.SemaphoreType.DMA,
        ],
    )
    return kfn(_k2_body)(md_flat, bl, genes_pad, cxg_pad, gix_pad, coord_pad,
                         frag_pad, rb_pad, ls, cells, rw)


def _k3_body(pm_ref, s_ref, h0_ref, h1_ref, latent_ref, rw_ref, rb_ref,
             ls_ref, out_ref):
    pm = pm_ref[...]
    sv = s_ref[...]
    r0 = lax.broadcasted_iota(jnp.int32, pm.shape, 0)
    c0 = lax.broadcasted_iota(jnp.int32, pm.shape, 1)
    maskc = (r0 * 128 + c0) < NC
    mix = jnp.sum(jnp.where(maskc, pm - jnp.log(jnp.where(maskc, sv, 1.0)),
                            0.0))
    mix = mix + jnp.float32(NC) * jnp.log(jnp.float32(C))

    fc = (h0_ref[...] + h1_ref[...]).astype(jnp.float32)
    rho = lax.dot_general(latent_ref[...], rw_ref[...],
                          (((1,), (1,)), ((), ())),
                          preferred_element_type=jnp.float32,
                          precision=lax.Precision.HIGHEST)
    fe = rb_ref[...] * jnp.exp(rho) * ls_ref[...]
    x = fc + 1.0
    z = x + 7.0
    prod = (x * (x + 1.0) * (x + 2.0) * (x + 3.0) * (x + 4.0) * (x + 5.0)
            * (x + 6.0))
    zi = 1.0 / z
    zi2 = zi * zi
    lg = ((z - 0.5) * jnp.log(z) - z + jnp.float32(0.9189385332046727)
          + zi * (jnp.float32(1.0 / 12.0)
                  - zi2 * (jnp.float32(1.0 / 360.0)
                           - zi2 * jnp.float32(1.0 / 1260.0)))
          - jnp.log(prod))
    lfc = fc * jnp.log(fe) - fe - lg
    out_ref[0, 0] = -(mix + jnp.sum(lfc))


def _k3(pm2, s2, h0, h1, latent, rwoi, rb_row, ls_col):
    return pl.pallas_call(
        _k3_body,
        out_shape=jax.ShapeDtypeStruct((1, 1), jnp.float32),
        out_specs=pl.BlockSpec(memory_space=pltpu.SMEM),
    )(pm2, s2, h0, h1, latent, rwoi, rb_row, ls_col)


def kernel(latent, genes_oi, cells_oi, cut_coordinates, cut_local_cellxgene_ix,
           cut_local_gene_ix, local_cellxgene_ix, n_cells, n_genes,
           logit_weight, rho_weight, bin_logit_baseline, rho_bias, libsize):
    genes_oi = genes_oi.astype(jnp.int32)
    cells_oi = cells_oi.astype(jnp.int32)
    cxg = cut_local_cellxgene_ix.astype(jnp.int32)
    gix = cut_local_gene_ix.astype(jnp.int32)
    frag = local_cellxgene_ix.astype(jnp.int32)

    md_flat = _k1(latent, genes_oi, logit_weight)

    genes_pad = jnp.pad(genes_oi, (0, 512 - G))
    cxg_pad = jnp.pad(cxg, (0, NCPAD - NC))
    gix_pad = jnp.pad(gix, (0, NCPAD - NC))
    coord_pad = jnp.pad(cut_coordinates, (0, NCPAD - NC))
    frag_pad = jnp.pad(frag, (0, NFPAD - NF), constant_values=BG)
    rb_pad = jnp.pad(rho_bias, (0, 5120 - NGT))
    rw_pad = jnp.pad(rho_weight, ((0, 0), (0, C - L)))

    pm, sv, hist, rboi, lsoi, rwoi = _k2(md_flat, bin_logit_baseline,
                                         genes_pad, cxg_pad, gix_pad,
                                         coord_pad, frag_pad, rb_pad,
                                         libsize, cells_oi, rw_pad)

    pm2 = pm.reshape(NCPAD // 128, 128)
    s2 = sv.reshape(NCPAD // 128, 128)
    h0 = hist[0, :BG].reshape(B, G)
    h1 = hist[1, :BG].reshape(B, G)
    rb_row = rboi[:G].reshape(1, G)
    ls_col = lsoi.reshape(B, 1)

    out = _k3(pm2, s2, h0, h1, latent, rwoi[:G, :L], rb_row, ls_col)
    scale = (jnp.asarray(n_cells, jnp.float32) * jnp.asarray(n_genes, jnp.float32)
             / jnp.float32(BG))
    return out[0, 0] * scale


# speedup vs baseline: 3.5274x; 1.2145x over previous
import functools

import jax
import jax.numpy as jnp
from jax import lax
from jax.experimental import pallas as pl
from jax.experimental.pallas import tpu as pltpu
from jax.experimental.pallas import tpu_sc as plsc

B = 256
G = 500
L = 32
C = 128
NGT = 5000
NC = 200000
NF = 400000
NW = 32
CH = 128
NC_CHUNKS = 1568
NCPAD = NC_CHUNKS * CH
NF_CHUNKS = 3136
NFPAD = NF_CHUNKS * CH
H = 131072
BG = B * G


GPS = 4


def _k1_body(genes_ref, latent_ref, lw_hbm, md_ref, lw_scr, sem):
    j = pl.program_id(0)
    latent = latent_ref[...]
    cps = [pltpu.make_async_copy(lw_hbm.at[genes_ref[j * GPS + k]],
                                 lw_scr.at[k], sem)
           for k in range(GPS)]
    for cp in cps:
        cp.start()
    for k in range(GPS):
        cps[k].wait()
        md_ref[pl.ds(k * B, B), :] = jnp.dot(
            latent, lw_scr[k], preferred_element_type=jnp.float32,
            precision=lax.Precision.HIGHEST)


def _k1(latent, genes_oi, logit_weight):
    grid_spec = pltpu.PrefetchScalarGridSpec(
        num_scalar_prefetch=1,
        grid=(G // GPS,),
        in_specs=[
            pl.BlockSpec((B, L), lambda g, gref: (0, 0)),
            pl.BlockSpec(memory_space=pl.ANY),
        ],
        out_specs=[
            pl.BlockSpec((GPS * B, C), lambda g, gref: (g, 0)),
        ],
        scratch_shapes=[
            pltpu.VMEM((GPS, L, C), jnp.float32),
            pltpu.SemaphoreType.DMA,
        ],
    )
    return pl.pallas_call(
        _k1_body,
        grid_spec=grid_spec,
        out_shape=[
            jax.ShapeDtypeStruct((G * B, C), jnp.float32),
        ],
    )(genes_oi, latent, logit_weight)[0]


def _k2_body(md_hbm, bl_hbm, genes_hbm, cxg_hbm, gix_hbm, coord_hbm,
             frag_hbm, rb_hbm, ls_hbm, cells_hbm, rw_hbm,
             pm_hbm, s_hbm, hist_hbm, rboi_hbm, lsoi_hbm, rwoi_hbm,
             genes_v, idx_all, gix_all, coord_all, pm_v, s_v,
             idx_a, gg_a, bins_a, md_a, bl_a,
             idx_b, gg_b, bins_b, md_b, bl_b,
             frag_a, frag_b, ones_v, zeros_v,
             rb_v, ls_v, cells_v, rboi_v, lsoi_v, gidx_v, rwoi_v,
             hist_sh, sem_md_a, sem_bl_a, sem_md_b, sem_bl_b,
             sem_f_a, sem_f_b, sem1):
    c = lax.axis_index("c")
    s = lax.axis_index("s")
    wid = s * 2 + c
    NJ = NC_CHUNKS // NW
    NJF = NF_CHUNKS // NW
    cbase = wid * (NJ * CH)
    fbase = wid * (NJF * CH)

    pltpu.sync_copy(genes_hbm, genes_v)
    pltpu.sync_copy(cxg_hbm.at[pl.ds(cbase, NJ * CH)], idx_all)
    pltpu.sync_copy(gix_hbm.at[pl.ds(cbase, NJ * CH)], gix_all)
    pltpu.sync_copy(coord_hbm.at[pl.ds(cbase, NJ * CH)], coord_all)

    @pl.when(s == 0)
    def _zero_hist():
        def zv(i, carry):
            zeros_v[pl.ds(i * 16, 16)] = jnp.zeros((16,), jnp.int32)
            return carry
        lax.fori_loop(0, 64, zv, 0)

        def zh(k, carry):
            pltpu.sync_copy(zeros_v, hist_sh.at[pl.ds(k * 1024, 1024)])
            return carry
        lax.fori_loop(0, H // 1024, zh, 0)

    for k in range(8):
        ones_v[pl.ds(k * 16, 16)] = jnp.full((16,), 1, jnp.int32)

    plsc.subcore_barrier()

    def ffire(j, fb, semf):
        toff = pl.multiple_of(fbase + j * CH, CH)
        pltpu.async_copy(frag_hbm.at[pl.ds(toff, CH)], fb, semf)

    ffire(0, frag_a, sem_f_a)

    def frag_step(j, carry):
        even = lax.rem(j, 2) == 0

        @pl.when(jnp.logical_and(even, j < NJF - 1))
        def _fb():
            ffire(j + 1, frag_b, sem_f_b)

        @pl.when(jnp.logical_and(jnp.logical_not(even), j < NJF - 1))
        def _fa():
            ffire(j + 1, frag_a, sem_f_a)

        @pl.when(even)
        def _sa():
            pltpu.make_async_copy(frag_hbm.at[pl.ds(0, CH)], frag_a,
                                  sem_f_a).wait()
            pltpu.sync_copy(ones_v, hist_sh.at[frag_a], add=True)

        @pl.when(jnp.logical_not(even))
        def _sb():
            pltpu.make_async_copy(frag_hbm.at[pl.ds(0, CH)], frag_b,
                                  sem_f_b).wait()
            pltpu.sync_copy(ones_v, hist_sh.at[frag_b], add=True)
        return carry
    lax.fori_loop(0, NJF, frag_step, 0)

    lane = lax.iota(jnp.int32, 16)
    lane0 = lane == 0

    def fire(j, idxb, ggb, binsb, mdb, blb, sem_md, sem_bl):
        for k in range(CH // 16):
            sl = pl.ds(k * 16, 16)
            gl = pl.ds(j * CH + k * 16, 16)
            ix = idx_all[gl]
            idxb[sl] = lax.rem(ix, jnp.int32(G)) * B + lax.div(ix, jnp.int32(G))
            ggb[sl] = plsc.load_gather(genes_v, [gix_all[gl]])
            b = (coord_all[gl] * jnp.float32(C)).astype(jnp.int32)
            binsb[sl] = jnp.clip(b, 0, C - 1)
        pltpu.async_copy(md_hbm.at[idxb], mdb, sem_md)
        pltpu.async_copy(bl_hbm.at[ggb], blb, sem_bl)

    def compute(j, idxb, ggb, binsb, mdb, blb, sem_md, sem_bl):
        pltpu.make_async_copy(md_hbm.at[idxb], mdb, sem_md).wait()
        pltpu.make_async_copy(bl_hbm.at[ggb], blb, sem_bl).wait()
        toff = pl.multiple_of(cbase + j * CH, CH)
        for k in range(CH // 16):
            sl = pl.ds(k * 16, 16)
            rows = lane + jnp.int32(k * 16)
            cols = binsb[sl]
            pm_v[sl] = (plsc.load_gather(mdb, [rows, cols])
                        + plsc.load_gather(blb, [rows, cols]))

        def cut_body(i, carry2):
            acc = jnp.zeros((16,), jnp.float32)
            for k in range(C // 16):
                sl = pl.ds(k * 16, 16)
                acc = acc + jnp.exp(mdb[i, sl] + blb[i, sl])
            sval = jnp.sum(acc)
            ii = jnp.full((16,), i, jnp.int32)
            plsc.store_scatter(s_v, [ii], sval + jnp.zeros((16,), jnp.float32),
                               mask=lane0)
            return carry2
        lax.fori_loop(0, CH, cut_body, 0)

        pltpu.sync_copy(pm_v, pm_hbm.at[pl.ds(toff, CH)])
        pltpu.sync_copy(s_v, s_hbm.at[pl.ds(toff, CH)])

    bufs_a = (idx_a, gg_a, bins_a, md_a, bl_a, sem_md_a, sem_bl_a)
    bufs_b = (idx_b, gg_b, bins_b, md_b, bl_b, sem_md_b, sem_bl_b)
    fire(0, *bufs_a)

    def cut_step(j, carry):
        even = lax.rem(j, 2) == 0

        @pl.when(jnp.logical_and(even, j < NJ - 1))
        def _fb():
            fire(j + 1, *bufs_b)

        @pl.when(jnp.logical_and(jnp.logical_not(even), j < NJ - 1))
        def _fa():
            fire(j + 1, *bufs_a)

        @pl.when(even)
        def _ca():
            compute(j, *bufs_a)

        @pl.when(jnp.logical_not(even))
        def _cb():
            compute(j, *bufs_b)
        return carry
    lax.fori_loop(0, NJ, cut_step, 0)

    @pl.when(jnp.logical_and(c == 0, s == 0))
    def _small_gathers():
        pltpu.sync_copy(rb_hbm, rb_v)
        pltpu.sync_copy(ls_hbm, ls_v)
        pltpu.sync_copy(cells_hbm, cells_v)
        for k in range(512 // 16):
            sl = pl.ds(k * 16, 16)
            rboi_v[sl] = plsc.load_gather(rb_v, [genes_v[sl]])
        for k in range(256 // 16):
            sl = pl.ds(k * 16, 16)
            lsoi_v[sl] = plsc.load_gather(ls_v, [cells_v[sl]])
        pltpu.sync_copy(rboi_v, rboi_hbm)
        pltpu.sync_copy(lsoi_v, lsoi_hbm)
        for k in range(512 // CH):
            pltpu.sync_copy(genes_hbm.at[pl.ds(k * CH, CH)], gidx_v)
            pltpu.async_copy(rw_hbm.at[gidx_v], rwoi_v, sem1).wait()
            pltpu.sync_copy(rwoi_v, rwoi_hbm.at[pl.ds(k * CH, CH)])

    plsc.subcore_barrier()

    @pl.when(s == 0)
    def _export_hist():
        pltpu.sync_copy(hist_sh, hist_hbm.at[c])


def _k2(md_flat, bl, genes_pad, cxg_pad, gix_pad, coord_pad, frag_pad,
        rb_pad, ls, cells, rw):
    kfn = functools.partial(
        pl.kernel,
        out_type=[
            jax.ShapeDtypeStruct((NCPAD,), jnp.float32),
            jax.ShapeDtypeStruct((NCPAD,), jnp.float32),
            jax.ShapeDtypeStruct((2, H), jnp.int32),
            jax.ShapeDtypeStruct((512,), jnp.float32),
            jax.ShapeDtypeStruct((256,), jnp.float32),
            jax.ShapeDtypeStruct((512, C), jnp.float32),
        ],
        mesh=plsc.VectorSubcoreMesh(core_axis_name="c", subcore_axis_name="s"),
        compiler_params=pltpu.CompilerParams(needs_layout_passes=False),
        scratch_types=[
            pltpu.VMEM((512,), jnp.int32),
            pltpu.VMEM((NCPAD // NW,), jnp.int32),
            pltpu.VMEM((NCPAD // NW,), jnp.int32),
            pltpu.VMEM((NCPAD // NW,), jnp.float32),
            pltpu.VMEM((CH,), jnp.float32),
            pltpu.VMEM((CH,), jnp.float32),
            pltpu.VMEM((CH,), jnp.int32),
            pltpu.VMEM((CH,), jnp.int32),
            pltpu.VMEM((CH,), jnp.int32),
            pltpu.VMEM((CH, C), jnp.float32),
            pltpu.VMEM((CH, C), jnp.float32),
            pltpu.VMEM((CH,), jnp.int32),
            pltpu.VMEM((CH,), jnp.int32),
            pltpu.VMEM((CH,), jnp.int32),
            pltpu.VMEM((CH, C), jnp.float32),
            pltpu.VMEM((CH, C), jnp.float32),
            pltpu.VMEM((CH,), jnp.int32),
            pltpu.VMEM((CH,), jnp.int32),
            pltpu.VMEM((CH,), jnp.int32),
            pltpu.VMEM((1024,), jnp.int32),
            pltpu.VMEM((5120,), jnp.float32),
            pltpu.VMEM((10000,), jnp.float32),
            pltpu.VMEM((256,), jnp.int32),
            pltpu.VMEM((512,), jnp.float32),
            pltpu.VMEM((256,), jnp.float32),
            pltpu.VMEM((CH,), jnp.int32),
            pltpu.VMEM((CH, C), jnp.float32),
            pltpu.VMEM_SHARED((H,), jnp.int32),
            pltpu.SemaphoreType.DMA,
            pltpu.SemaphoreType.DMA,
            pltpu.SemaphoreType.DMA,
            pltpu.SemaphoreType.DMA,
            pltpu.SemaphoreType.DMA,
            pltpu.SemaphoreType.DMA,
            pltpu.SemaphoreType.DMA,
        ],
    )
    return kfn(_k2_body)(md_flat, bl, genes_pad, cxg_pad, gix_pad, coord_pad,
                         frag_pad, rb_pad, ls, cells, rw)


def _k3_body(pm_ref, s_ref, h0_ref, h1_ref, latent_ref, rw_ref, rb_ref,
             ls_ref, out_ref):
    pm = pm_ref[...]
    sv = s_ref[...]
    r0 = lax.broadcasted_iota(jnp.int32, pm.shape, 0)
    c0 = lax.broadcasted_iota(jnp.int32, pm.shape, 1)
    maskc = (r0 * 128 + c0) < NC
    mix = jnp.sum(jnp.where(maskc, pm - jnp.log(jnp.where(maskc, sv, 1.0)),
                            0.0))
    mix = mix + jnp.float32(NC) * jnp.log(jnp.float32(C))

    fc = (h0_ref[...] + h1_ref[...]).astype(jnp.float32)
    rho = lax.dot_general(latent_ref[...], rw_ref[...],
                          (((1,), (1,)), ((), ())),
                          preferred_element_type=jnp.float32,
                          precision=lax.Precision.HIGHEST)
    fe = rb_ref[...] * jnp.exp(rho) * ls_ref[...]
    x = fc + 1.0
    z = x + 7.0
    prod = (x * (x + 1.0) * (x + 2.0) * (x + 3.0) * (x + 4.0) * (x + 5.0)
            * (x + 6.0))
    zi = 1.0 / z
    zi2 = zi * zi
    lg = ((z - 0.5) * jnp.log(z) - z + jnp.float32(0.9189385332046727)
          + zi * (jnp.float32(1.0 / 12.0)
                  - zi2 * (jnp.float32(1.0 / 360.0)
                           - zi2 * jnp.float32(1.0 / 1260.0)))
          - jnp.log(prod))
    lfc = fc * jnp.log(fe) - fe - lg
    out_ref[0, 0] = -(mix + jnp.sum(lfc))


def _k3(pm2, s2, h0, h1, latent, rwoi, rb_row, ls_col):
    return pl.pallas_call(
        _k3_body,
        out_shape=jax.ShapeDtypeStruct((1, 1), jnp.float32),
        out_specs=pl.BlockSpec(memory_space=pltpu.SMEM),
    )(pm2, s2, h0, h1, latent, rwoi, rb_row, ls_col)


def kernel(latent, genes_oi, cells_oi, cut_coordinates, cut_local_cellxgene_ix,
           cut_local_gene_ix, local_cellxgene_ix, n_cells, n_genes,
           logit_weight, rho_weight, bin_logit_baseline, rho_bias, libsize):
    genes_oi = genes_oi.astype(jnp.int32)
    cells_oi = cells_oi.astype(jnp.int32)
    cxg = cut_local_cellxgene_ix.astype(jnp.int32)
    gix = cut_local_gene_ix.astype(jnp.int32)
    frag = local_cellxgene_ix.astype(jnp.int32)

    md_flat = _k1(latent, genes_oi, logit_weight)

    genes_pad = jnp.pad(genes_oi, (0, 512 - G))
    cxg_pad = jnp.pad(cxg, (0, NCPAD - NC))
    gix_pad = jnp.pad(gix, (0, NCPAD - NC))
    coord_pad = jnp.pad(cut_coordinates, (0, NCPAD - NC))
    frag_pad = jnp.pad(frag, (0, NFPAD - NF), constant_values=BG)
    rb_pad = jnp.pad(rho_bias, (0, 5120 - NGT))
    rw_pad = jnp.pad(rho_weight, ((0, 0), (0, C - L)))

    pm, sv, hist, rboi, lsoi, rwoi = _k2(md_flat, bin_logit_baseline,
                                         genes_pad, cxg_pad, gix_pad,
                                         coord_pad, frag_pad, rb_pad,
                                         libsize, cells_oi, rw_pad)

    pm2 = pm.reshape(NCPAD // 128, 128)
    s2 = sv.reshape(NCPAD // 128, 128)
    h0 = hist[0, :BG].reshape(B, G)
    h1 = hist[1, :BG].reshape(B, G)
    rb_row = rboi[:G].reshape(1, G)
    ls_col = lsoi.reshape(B, 1)

    out = _k3(pm2, s2, h0, h1, latent, rwoi[:G, :L], rb_row, ls_col)
    scale = (jnp.asarray(n_cells, jnp.float32) * jnp.asarray(n_genes, jnp.float32)
             / jnp.float32(BG))
    return out[0, 0] * scale


# speedup vs baseline: 4.8844x; 1.3847x over previous
import functools

import jax
import jax.numpy as jnp
from jax import lax
from jax.experimental import pallas as pl
from jax.experimental.pallas import tpu as pltpu
from jax.experimental.pallas import tpu_sc as plsc

B = 256
G = 500
L = 32
C = 128
NGT = 5000
NC = 200000
NF = 400000
NW = 32
CH = 128
NC_CHUNKS = 1568
NCPAD = NC_CHUNKS * CH
NF_CHUNKS = 3136
NFPAD = NF_CHUNKS * CH
H = 131072
BG = B * G


GPS = 4


def _k1_body(genes_ref, latent_ref, lw_hbm, md_ref, lw_scr, sem):
    j = pl.program_id(0)
    nsteps = pl.num_programs(0)
    latent = latent_ref[...]

    def fire(jj, slot):
        for k in range(GPS):
            pltpu.make_async_copy(lw_hbm.at[genes_ref[jj * GPS + k]],
                                  lw_scr.at[slot, k], sem).start()

    @pl.when(j == 0)
    def _prime():
        fire(0, 0)

    @pl.when(j < nsteps - 1)
    def _next():
        fire(j + 1, lax.rem(j + 1, 2))

    slot = lax.rem(j, 2)
    for k in range(GPS):
        pltpu.make_async_copy(lw_hbm.at[genes_ref[j * GPS + k]],
                              lw_scr.at[slot, k], sem).wait()
        md_ref[pl.ds(k * B, B), :] = jnp.dot(
            latent, lw_scr[slot, k], preferred_element_type=jnp.float32)


def _k1(latent, genes_oi, logit_weight):
    grid_spec = pltpu.PrefetchScalarGridSpec(
        num_scalar_prefetch=1,
        grid=(G // GPS,),
        in_specs=[
            pl.BlockSpec((B, L), lambda g, gref: (0, 0)),
            pl.BlockSpec(memory_space=pl.ANY),
        ],
        out_specs=[
            pl.BlockSpec((GPS * B, C), lambda g, gref: (g, 0)),
        ],
        scratch_shapes=[
            pltpu.VMEM((2, GPS, L, C), jnp.float32),
            pltpu.SemaphoreType.DMA,
        ],
    )
    return pl.pallas_call(
        _k1_body,
        grid_spec=grid_spec,
        out_shape=[
            jax.ShapeDtypeStruct((G * B, C), jnp.float32),
        ],
    )(genes_oi, latent, logit_weight)[0]


def _k2_body(md_hbm, bl_hbm, genes_hbm, cxg_hbm, gix_hbm, coord_hbm,
             frag_hbm, rb_hbm, ls_hbm, cells_hbm, rw_hbm,
             pm_hbm, s_hbm, hist_hbm, rboi_hbm, lsoi_hbm, rwoi_hbm,
             genes_v, idx_all, gix_all, coord_all, pm_v, s_v,
             idx_a, gg_a, bins_a, md_a, bl_a,
             idx_b, gg_b, bins_b, md_b, bl_b,
             frag_a, frag_b, ones_v, zeros_v,
             rb_v, ls_v, cells_v, rboi_v, lsoi_v, gidx_v, rwoi_v,
             hist_sh, sem_md_a, sem_bl_a, sem_md_b, sem_bl_b,
             sem_f_a, sem_f_b, sem1):
    c = lax.axis_index("c")
    s = lax.axis_index("s")
    wid = s * 2 + c
    NJ = NC_CHUNKS // NW
    NJF = NF_CHUNKS // NW
    cbase = wid * (NJ * CH)
    fbase = wid * (NJF * CH)

    pltpu.sync_copy(genes_hbm, genes_v)
    pltpu.sync_copy(cxg_hbm.at[pl.ds(cbase, NJ * CH)], idx_all)
    pltpu.sync_copy(gix_hbm.at[pl.ds(cbase, NJ * CH)], gix_all)
    pltpu.sync_copy(coord_hbm.at[pl.ds(cbase, NJ * CH)], coord_all)

    @pl.when(s == 0)
    def _zero_hist():
        def zv(i, carry):
            zeros_v[pl.ds(i * 16, 16)] = jnp.zeros((16,), jnp.int32)
            return carry
        lax.fori_loop(0, 64, zv, 0)

        def zh(k, carry):
            pltpu.sync_copy(zeros_v, hist_sh.at[pl.ds(k * 1024, 1024)])
            return carry
        lax.fori_loop(0, H // 1024, zh, 0)

    for k in range(8):
        ones_v[pl.ds(k * 16, 16)] = jnp.full((16,), 1, jnp.int32)

    plsc.subcore_barrier()

    def ffire(j, fb, semf):
        toff = pl.multiple_of(fbase + j * CH, CH)
        pltpu.async_copy(frag_hbm.at[pl.ds(toff, CH)], fb, semf)

    ffire(0, frag_a, sem_f_a)

    def frag_step(j, carry):
        even = lax.rem(j, 2) == 0

        @pl.when(jnp.logical_and(even, j < NJF - 1))
        def _fb():
            ffire(j + 1, frag_b, sem_f_b)

        @pl.when(jnp.logical_and(jnp.logical_not(even), j < NJF - 1))
        def _fa():
            ffire(j + 1, frag_a, sem_f_a)

        @pl.when(even)
        def _sa():
            pltpu.make_async_copy(frag_hbm.at[pl.ds(0, CH)], frag_a,
                                  sem_f_a).wait()
            pltpu.sync_copy(ones_v, hist_sh.at[frag_a], add=True)

        @pl.when(jnp.logical_not(even))
        def _sb():
            pltpu.make_async_copy(frag_hbm.at[pl.ds(0, CH)], frag_b,
                                  sem_f_b).wait()
            pltpu.sync_copy(ones_v, hist_sh.at[frag_b], add=True)
        return carry
    lax.fori_loop(0, NJF, frag_step, 0)

    lane = lax.iota(jnp.int32, 16)
    lane0 = lane == 0

    def fire(j, idxb, ggb, binsb, mdb, blb, sem_md, sem_bl):
        for k in range(CH // 16):
            sl = pl.ds(k * 16, 16)
            gl = pl.ds(j * CH + k * 16, 16)
            ix = idx_all[gl]
            idxb[sl] = lax.rem(ix, jnp.int32(G)) * B + lax.div(ix, jnp.int32(G))
            ggb[sl] = plsc.load_gather(genes_v, [gix_all[gl]])
            b = (coord_all[gl] * jnp.float32(C)).astype(jnp.int32)
            binsb[sl] = jnp.clip(b, 0, C - 1)
        pltpu.async_copy(md_hbm.at[idxb], mdb, sem_md)
        pltpu.async_copy(bl_hbm.at[ggb], blb, sem_bl)

    def compute(j, idxb, ggb, binsb, mdb, blb, sem_md, sem_bl):
        pltpu.make_async_copy(md_hbm.at[idxb], mdb, sem_md).wait()
        pltpu.make_async_copy(bl_hbm.at[ggb], blb, sem_bl).wait()
        toff = pl.multiple_of(cbase + j * CH, CH)
        for k in range(CH // 16):
            sl = pl.ds(k * 16, 16)
            rows = lane + jnp.int32(k * 16)
            cols = binsb[sl]
            pm_v[sl] = (plsc.load_gather(mdb, [rows, cols])
                        + plsc.load_gather(blb, [rows, cols]))

        def cut_body(i, carry2):
            acc = jnp.zeros((16,), jnp.float32)
            for k in range(C // 16):
                sl = pl.ds(k * 16, 16)
                acc = acc + jnp.exp(mdb[i, sl] + blb[i, sl])
            sval = jnp.sum(acc)
            ii = jnp.full((16,), i, jnp.int32)
            plsc.store_scatter(s_v, [ii], sval + jnp.zeros((16,), jnp.float32),
                               mask=lane0)
            return carry2
        lax.fori_loop(0, CH, cut_body, 0)

        pltpu.sync_copy(pm_v, pm_hbm.at[pl.ds(toff, CH)])
        pltpu.sync_copy(s_v, s_hbm.at[pl.ds(toff, CH)])

    bufs_a = (idx_a, gg_a, bins_a, md_a, bl_a, sem_md_a, sem_bl_a)
    bufs_b = (idx_b, gg_b, bins_b, md_b, bl_b, sem_md_b, sem_bl_b)
    fire(0, *bufs_a)

    def cut_step(j, carry):
        even = lax.rem(j, 2) == 0

        @pl.when(jnp.logical_and(even, j < NJ - 1))
        def _fb():
            fire(j + 1, *bufs_b)

        @pl.when(jnp.logical_and(jnp.logical_not(even), j < NJ - 1))
        def _fa():
            fire(j + 1, *bufs_a)

        @pl.when(even)
        def _ca():
            compute(j, *bufs_a)

        @pl.when(jnp.logical_not(even))
        def _cb():
            compute(j, *bufs_b)
        return carry
    lax.fori_loop(0, NJ, cut_step, 0)

    @pl.when(jnp.logical_and(c == 0, s == 0))
    def _small_gathers():
        pltpu.sync_copy(rb_hbm, rb_v)
        pltpu.sync_copy(ls_hbm, ls_v)
        pltpu.sync_copy(cells_hbm, cells_v)
        for k in range(512 // 16):
            sl = pl.ds(k * 16, 16)
            rboi_v[sl] = plsc.load_gather(rb_v, [genes_v[sl]])
        for k in range(256 // 16):
            sl = pl.ds(k * 16, 16)
            lsoi_v[sl] = plsc.load_gather(ls_v, [cells_v[sl]])
        pltpu.sync_copy(rboi_v, rboi_hbm)
        pltpu.sync_copy(lsoi_v, lsoi_hbm)
        for k in range(512 // CH):
            pltpu.sync_copy(genes_hbm.at[pl.ds(k * CH, CH)], gidx_v)
            pltpu.async_copy(rw_hbm.at[gidx_v], rwoi_v, sem1).wait()
            pltpu.sync_copy(rwoi_v, rwoi_hbm.at[pl.ds(k * CH, CH)])

    plsc.subcore_barrier()

    @pl.when(s == 0)
    def _export_hist():
        pltpu.sync_copy(hist_sh, hist_hbm.at[c])


def _k2(md_flat, bl, genes_pad, cxg_pad, gix_pad, coord_pad, frag_pad,
        rb_pad, ls, cells, rw):
    kfn = functools.partial(
        pl.kernel,
        out_type=[
            jax.ShapeDtypeStruct((NCPAD,), jnp.float32),
            jax.ShapeDtypeStruct((NCPAD,), jnp.float32),
            jax.ShapeDtypeStruct((2, H), jnp.int32),
            jax.ShapeDtypeStruct((512,), jnp.float32),
            jax.ShapeDtypeStruct((256,), jnp.float32),
            jax.ShapeDtypeStruct((512, C), jnp.float32),
        ],
        mesh=plsc.VectorSubcoreMesh(core_axis_name="c", subcore_axis_name="s"),
        compiler_params=pltpu.CompilerParams(needs_layout_passes=False),
        scratch_types=[
            pltpu.VMEM((512,), jnp.int32),
            pltpu.VMEM((NCPAD // NW,), jnp.int32),
            pltpu.VMEM((NCPAD // NW,), jnp.int32),
            pltpu.VMEM((NCPAD // NW,), jnp.float32),
            pltpu.VMEM((CH,), jnp.float32),
            pltpu.VMEM((CH,), jnp.float32),
            pltpu.VMEM((CH,), jnp.int32),
            pltpu.VMEM((CH,), jnp.int32),
            pltpu.VMEM((CH,), jnp.int32),
            pltpu.VMEM((CH, C), jnp.float32),
            pltpu.VMEM((CH, C), jnp.float32),
            pltpu.VMEM((CH,), jnp.int32),
            pltpu.VMEM((CH,), jnp.int32),
            pltpu.VMEM((CH,), jnp.int32),
            pltpu.VMEM((CH, C), jnp.float32),
            pltpu.VMEM((CH, C), jnp.float32),
            pltpu.VMEM((CH,), jnp.int32),
            pltpu.VMEM((CH,), jnp.int32),
            pltpu.VMEM((CH,), jnp.int32),
            pltpu.VMEM((1024,), jnp.int32),
            pltpu.VMEM((5120,), jnp.float32),
            pltpu.VMEM((10000,), jnp.float32),
            pltpu.VMEM((256,), jnp.int32),
            pltpu.VMEM((512,), jnp.float32),
            pltpu.VMEM((256,), jnp.float32),
            pltpu.VMEM((CH,), jnp.int32),
            pltpu.VMEM((CH, C), jnp.float32),
            pltpu.VMEM_SHARED((H,), jnp.int32),
            pltpu.SemaphoreType.DMA,
            pltpu.SemaphoreType.DMA,
            pltpu.SemaphoreType.DMA,
            pltpu.SemaphoreType.DMA,
            pltpu.SemaphoreType.DMA,
            pltpu.SemaphoreType.DMA,
            pltpu.SemaphoreType.DMA,
        ],
    )
    return kfn(_k2_body)(md_flat, bl, genes_pad, cxg_pad, gix_pad, coord_pad,
                         frag_pad, rb_pad, ls, cells, rw)


def _k3_body(pm_ref, s_ref, h0_ref, h1_ref, latent_ref, rw_ref, rb_ref,
             ls_ref, out_ref):
    pm = pm_ref[...]
    sv = s_ref[...]
    r0 = lax.broadcasted_iota(jnp.int32, pm.shape, 0)
    c0 = lax.broadcasted_iota(jnp.int32, pm.shape, 1)
    maskc = (r0 * 128 + c0) < NC
    mix = jnp.sum(jnp.where(maskc, pm - jnp.log(jnp.where(maskc, sv, 1.0)),
                            0.0))
    mix = mix + jnp.float32(NC) * jnp.log(jnp.float32(C))

    fc = (h0_ref[...] + h1_ref[...]).astype(jnp.float32)
    rho = lax.dot_general(latent_ref[...], rw_ref[...],
                          (((1,), (1,)), ((), ())),
                          preferred_element_type=jnp.float32,
                          precision=lax.Precision.HIGHEST)
    fe = rb_ref[...] * jnp.exp(rho) * ls_ref[...]
    x = fc + 1.0
    z = x + 7.0
    prod = (x * (x + 1.0) * (x + 2.0) * (x + 3.0) * (x + 4.0) * (x + 5.0)
            * (x + 6.0))
    zi = 1.0 / z
    zi2 = zi * zi
    lg = ((z - 0.5) * jnp.log(z) - z + jnp.float32(0.9189385332046727)
          + zi * (jnp.float32(1.0 / 12.0)
                  - zi2 * (jnp.float32(1.0 / 360.0)
                           - zi2 * jnp.float32(1.0 / 1260.0)))
          - jnp.log(prod))
    lfc = fc * jnp.log(fe) - fe - lg
    out_ref[0, 0] = -(mix + jnp.sum(lfc))


def _k3(pm2, s2, h0, h1, latent, rwoi, rb_row, ls_col):
    return pl.pallas_call(
        _k3_body,
        out_shape=jax.ShapeDtypeStruct((1, 1), jnp.float32),
        out_specs=pl.BlockSpec(memory_space=pltpu.SMEM),
    )(pm2, s2, h0, h1, latent, rwoi, rb_row, ls_col)


def kernel(latent, genes_oi, cells_oi, cut_coordinates, cut_local_cellxgene_ix,
           cut_local_gene_ix, local_cellxgene_ix, n_cells, n_genes,
           logit_weight, rho_weight, bin_logit_baseline, rho_bias, libsize):
    genes_oi = genes_oi.astype(jnp.int32)
    cells_oi = cells_oi.astype(jnp.int32)
    cxg = cut_local_cellxgene_ix.astype(jnp.int32)
    gix = cut_local_gene_ix.astype(jnp.int32)
    frag = local_cellxgene_ix.astype(jnp.int32)

    md_flat = _k1(latent, genes_oi, logit_weight)

    genes_pad = jnp.pad(genes_oi, (0, 512 - G))
    cxg_pad = jnp.pad(cxg, (0, NCPAD - NC))
    gix_pad = jnp.pad(gix, (0, NCPAD - NC))
    coord_pad = jnp.pad(cut_coordinates, (0, NCPAD - NC))
    frag_pad = jnp.pad(frag, (0, NFPAD - NF), constant_values=BG)
    rb_pad = jnp.pad(rho_bias, (0, 5120 - NGT))
    rw_pad = jnp.pad(rho_weight, ((0, 0), (0, C - L)))

    pm, sv, hist, rboi, lsoi, rwoi = _k2(md_flat, bin_logit_baseline,
                                         genes_pad, cxg_pad, gix_pad,
                                         coord_pad, frag_pad, rb_pad,
                                         libsize, cells_oi, rw_pad)

    pm2 = pm.reshape(NCPAD // 128, 128)
    s2 = sv.reshape(NCPAD // 128, 128)
    h0 = hist[0, :BG].reshape(B, G)
    h1 = hist[1, :BG].reshape(B, G)
    rb_row = rboi[:G].reshape(1, G)
    ls_col = lsoi.reshape(B, 1)

    out = _k3(pm2, s2, h0, h1, latent, rwoi[:G, :L], rb_row, ls_col)
    scale = (jnp.asarray(n_cells, jnp.float32) * jnp.asarray(n_genes, jnp.float32)
             / jnp.float32(BG))
    return out[0, 0] * scale


# speedup vs baseline: 5.4257x; 1.1108x over previous
import functools

import jax
import jax.numpy as jnp
from jax import lax
from jax.experimental import pallas as pl
from jax.experimental.pallas import tpu as pltpu
from jax.experimental.pallas import tpu_sc as plsc

B = 256
G = 500
L = 32
C = 128
NGT = 5000
NC = 200000
NF = 400000
NW = 32
CH = 128
NC_CHUNKS = 1568
NCPAD = NC_CHUNKS * CH
NF_CHUNKS = 3136
NFPAD = NF_CHUNKS * CH
H = 131072
BG = B * G


GPS = 4


def _k1_body(genes_ref, latent_ref, lw_hbm, md_ref, lw_scr, sem):
    j = pl.program_id(0)
    nsteps = pl.num_programs(0)
    latent = latent_ref[...]

    def fire(jj, slot):
        for k in range(GPS):
            pltpu.make_async_copy(lw_hbm.at[genes_ref[jj * GPS + k]],
                                  lw_scr.at[slot, k], sem).start()

    @pl.when(j == 0)
    def _prime():
        fire(0, 0)

    @pl.when(j < nsteps - 1)
    def _next():
        fire(j + 1, lax.rem(j + 1, 2))

    slot = lax.rem(j, 2)
    for k in range(GPS):
        pltpu.make_async_copy(lw_hbm.at[genes_ref[j * GPS + k]],
                              lw_scr.at[slot, k], sem).wait()
        md_ref[pl.ds(k * B, B), :] = jnp.dot(
            latent, lw_scr[slot, k], preferred_element_type=jnp.float32)


def _k1(latent, genes_oi, logit_weight):
    grid_spec = pltpu.PrefetchScalarGridSpec(
        num_scalar_prefetch=1,
        grid=(G // GPS,),
        in_specs=[
            pl.BlockSpec((B, L), lambda g, gref: (0, 0)),
            pl.BlockSpec(memory_space=pl.ANY),
        ],
        out_specs=[
            pl.BlockSpec((GPS * B, C), lambda g, gref: (g, 0)),
        ],
        scratch_shapes=[
            pltpu.VMEM((2, GPS, L, C), jnp.float32),
            pltpu.SemaphoreType.DMA,
        ],
    )
    return pl.pallas_call(
        _k1_body,
        grid_spec=grid_spec,
        out_shape=[
            jax.ShapeDtypeStruct((G * B, C), jnp.float32),
        ],
    )(genes_oi, latent, logit_weight)[0]


def _k2a_body(genes_hbm, frag_hbm, rb_hbm, ls_hbm, cells_hbm, rw_hbm,
              hist_hbm, rboi_hbm, lsoi_hbm, rwoi_hbm,
              genes_v, frag_a, frag_b, ones_v, zeros_v,
              rb_v, ls_v, cells_v, rboi_v, lsoi_v, gidx_v, rwoi_v,
              hist_sh, sem_f_a, sem_f_b, sem1):
    c = lax.axis_index("c")
    s = lax.axis_index("s")
    wid = s * 2 + c
    NJF = NF_CHUNKS // NW
    fbase = wid * (NJF * CH)

    pltpu.sync_copy(genes_hbm, genes_v)

    @pl.when(s == 0)
    def _zero_hist():
        def zv(i, carry):
            zeros_v[pl.ds(i * 16, 16)] = jnp.zeros((16,), jnp.int32)
            return carry
        lax.fori_loop(0, 64, zv, 0)

        def zh(k, carry):
            pltpu.sync_copy(zeros_v, hist_sh.at[pl.ds(k * 1024, 1024)])
            return carry
        lax.fori_loop(0, H // 1024, zh, 0)

    for k in range(8):
        ones_v[pl.ds(k * 16, 16)] = jnp.full((16,), 1, jnp.int32)

    plsc.subcore_barrier()

    def ffire(j, fb, semf):
        toff = pl.multiple_of(fbase + j * CH, CH)
        pltpu.async_copy(frag_hbm.at[pl.ds(toff, CH)], fb, semf)

    ffire(0, frag_a, sem_f_a)

    def frag_step(j, carry):
        even = lax.rem(j, 2) == 0

        @pl.when(jnp.logical_and(even, j < NJF - 1))
        def _fb():
            ffire(j + 1, frag_b, sem_f_b)

        @pl.when(jnp.logical_and(jnp.logical_not(even), j < NJF - 1))
        def _fa():
            ffire(j + 1, frag_a, sem_f_a)

        @pl.when(even)
        def _sa():
            pltpu.make_async_copy(frag_hbm.at[pl.ds(0, CH)], frag_a,
                                  sem_f_a).wait()
            pltpu.sync_copy(ones_v, hist_sh.at[frag_a], add=True)

        @pl.when(jnp.logical_not(even))
        def _sb():
            pltpu.make_async_copy(frag_hbm.at[pl.ds(0, CH)], frag_b,
                                  sem_f_b).wait()
            pltpu.sync_copy(ones_v, hist_sh.at[frag_b], add=True)
        return carry
    lax.fori_loop(0, NJF, frag_step, 0)

    @pl.when(jnp.logical_and(c == 0, s == 0))
    def _small_gathers():
        pltpu.sync_copy(rb_hbm, rb_v)
        pltpu.sync_copy(ls_hbm, ls_v)
        pltpu.sync_copy(cells_hbm, cells_v)
        for k in range(512 // 16):
            sl = pl.ds(k * 16, 16)
            rboi_v[sl] = plsc.load_gather(rb_v, [genes_v[sl]])
        for k in range(256 // 16):
            sl = pl.ds(k * 16, 16)
            lsoi_v[sl] = plsc.load_gather(ls_v, [cells_v[sl]])
        pltpu.sync_copy(rboi_v, rboi_hbm)
        pltpu.sync_copy(lsoi_v, lsoi_hbm)
        for k in range(512 // CH):
            pltpu.sync_copy(genes_hbm.at[pl.ds(k * CH, CH)], gidx_v)
            pltpu.async_copy(rw_hbm.at[gidx_v], rwoi_v, sem1).wait()
            pltpu.sync_copy(rwoi_v, rwoi_hbm.at[pl.ds(k * CH, CH)])

    plsc.subcore_barrier()

    @pl.when(s == 0)
    def _export_hist():
        pltpu.sync_copy(hist_sh, hist_hbm.at[c])


def _k2a(genes_pad, frag_pad, rb_pad, ls, cells, rw):
    kfn = functools.partial(
        pl.kernel,
        out_type=[
            jax.ShapeDtypeStruct((2, H), jnp.int32),
            jax.ShapeDtypeStruct((512,), jnp.float32),
            jax.ShapeDtypeStruct((256,), jnp.float32),
            jax.ShapeDtypeStruct((512, C), jnp.float32),
        ],
        mesh=plsc.VectorSubcoreMesh(core_axis_name="c", subcore_axis_name="s"),
        compiler_params=pltpu.CompilerParams(needs_layout_passes=False),
        scratch_types=[
            pltpu.VMEM((512,), jnp.int32),
            pltpu.VMEM((CH,), jnp.int32),
            pltpu.VMEM((CH,), jnp.int32),
            pltpu.VMEM((CH,), jnp.int32),
            pltpu.VMEM((1024,), jnp.int32),
            pltpu.VMEM((5120,), jnp.float32),
            pltpu.VMEM((10000,), jnp.float32),
            pltpu.VMEM((256,), jnp.int32),
            pltpu.VMEM((512,), jnp.float32),
            pltpu.VMEM((256,), jnp.float32),
            pltpu.VMEM((CH,), jnp.int32),
            pltpu.VMEM((CH, C), jnp.float32),
            pltpu.VMEM_SHARED((H,), jnp.int32),
            pltpu.SemaphoreType.DMA,
            pltpu.SemaphoreType.DMA,
            pltpu.SemaphoreType.DMA,
        ],
    )
    return kfn(_k2a_body)(genes_pad, frag_pad, rb_pad, ls, cells, rw)


def _k2b_body(md_hbm, bl_hbm, genes_hbm, cxg_hbm, gix_hbm, coord_hbm,
              pm_hbm, s_hbm,
              genes_v, idx_all, gix_all, coord_all, pm_v, s_v,
              idx_a, gg_a, bins_a, md_a, bl_a,
              idx_b, gg_b, bins_b, md_b, bl_b,
              sem_md_a, sem_bl_a, sem_md_b, sem_bl_b):
    c = lax.axis_index("c")
    s = lax.axis_index("s")
    wid = s * 2 + c
    NJ = NC_CHUNKS // NW
    cbase = wid * (NJ * CH)

    pltpu.sync_copy(genes_hbm, genes_v)
    pltpu.sync_copy(cxg_hbm.at[pl.ds(cbase, NJ * CH)], idx_all)
    pltpu.sync_copy(gix_hbm.at[pl.ds(cbase, NJ * CH)], gix_all)
    pltpu.sync_copy(coord_hbm.at[pl.ds(cbase, NJ * CH)], coord_all)

    lane = lax.iota(jnp.int32, 16)
    lane0 = lane == 0

    def fire(j, idxb, ggb, binsb, mdb, blb, sem_md, sem_bl):
        for k in range(CH // 16):
            sl = pl.ds(k * 16, 16)
            gl = pl.ds(j * CH + k * 16, 16)
            ix = idx_all[gl]
            idxb[sl] = lax.rem(ix, jnp.int32(G)) * B + lax.div(ix, jnp.int32(G))
            ggb[sl] = plsc.load_gather(genes_v, [gix_all[gl]])
            b = (coord_all[gl] * jnp.float32(C)).astype(jnp.int32)
            binsb[sl] = jnp.clip(b, 0, C - 1)
        pltpu.async_copy(md_hbm.at[idxb], mdb, sem_md)
        pltpu.async_copy(bl_hbm.at[ggb], blb, sem_bl)

    def compute(j, idxb, ggb, binsb, mdb, blb, sem_md, sem_bl):
        pltpu.make_async_copy(md_hbm.at[idxb], mdb, sem_md).wait()
        pltpu.make_async_copy(bl_hbm.at[ggb], blb, sem_bl).wait()
        toff = pl.multiple_of(cbase + j * CH, CH)
        for k in range(CH // 16):
            sl = pl.ds(k * 16, 16)
            rows = lane + jnp.int32(k * 16)
            cols = binsb[sl]
            pm_v[sl] = (plsc.load_gather(mdb, [rows, cols])
                        + plsc.load_gather(blb, [rows, cols]))

        def cut_body(i, carry2):
            acc = jnp.zeros((16,), jnp.float32)
            for k in range(C // 16):
                sl = pl.ds(k * 16, 16)
                acc = acc + jnp.exp(mdb[i, sl] + blb[i, sl])
            sval = jnp.sum(acc)
            ii = jnp.full((16,), i, jnp.int32)
            plsc.store_scatter(s_v, [ii], sval + jnp.zeros((16,), jnp.float32),
                               mask=lane0)
            return carry2
        lax.fori_loop(0, CH, cut_body, 0)

        pltpu.sync_copy(pm_v, pm_hbm.at[pl.ds(toff, CH)])
        pltpu.sync_copy(s_v, s_hbm.at[pl.ds(toff, CH)])

    bufs_a = (idx_a, gg_a, bins_a, md_a, bl_a, sem_md_a, sem_bl_a)
    bufs_b = (idx_b, gg_b, bins_b, md_b, bl_b, sem_md_b, sem_bl_b)
    fire(0, *bufs_a)

    def cut_step(j, carry):
        even = lax.rem(j, 2) == 0

        @pl.when(jnp.logical_and(even, j < NJ - 1))
        def _fb():
            fire(j + 1, *bufs_b)

        @pl.when(jnp.logical_and(jnp.logical_not(even), j < NJ - 1))
        def _fa():
            fire(j + 1, *bufs_a)

        @pl.when(even)
        def _ca():
            compute(j, *bufs_a)

        @pl.when(jnp.logical_not(even))
        def _cb():
            compute(j, *bufs_b)
        return carry
    lax.fori_loop(0, NJ, cut_step, 0)


def _k2b(md_flat, bl, genes_pad, cxg_pad, gix_pad, coord_pad):
    kfn = functools.partial(
        pl.kernel,
        out_type=[
            jax.ShapeDtypeStruct((NCPAD,), jnp.float32),
            jax.ShapeDtypeStruct((NCPAD,), jnp.float32),
        ],
        mesh=plsc.VectorSubcoreMesh(core_axis_name="c", subcore_axis_name="s"),
        compiler_params=pltpu.CompilerParams(needs_layout_passes=False),
        scratch_types=[
            pltpu.VMEM((512,), jnp.int32),
            pltpu.VMEM((NCPAD // NW,), jnp.int32),
            pltpu.VMEM((NCPAD // NW,), jnp.int32),
            pltpu.VMEM((NCPAD // NW,), jnp.float32),
            pltpu.VMEM((CH,), jnp.float32),
            pltpu.VMEM((CH,), jnp.float32),
            pltpu.VMEM((CH,), jnp.int32),
            pltpu.VMEM((CH,), jnp.int32),
            pltpu.VMEM((CH,), jnp.int32),
            pltpu.VMEM((CH, C), jnp.float32),
            pltpu.VMEM((CH, C), jnp.float32),
            pltpu.VMEM((CH,), jnp.int32),
            pltpu.VMEM((CH,), jnp.int32),
            pltpu.VMEM((CH,), jnp.int32),
            pltpu.VMEM((CH, C), jnp.float32),
            pltpu.VMEM((CH, C), jnp.float32),
            pltpu.SemaphoreType.DMA,
            pltpu.SemaphoreType.DMA,
            pltpu.SemaphoreType.DMA,
            pltpu.SemaphoreType.DMA,
        ],
    )
    return kfn(_k2b_body)(md_flat, bl, genes_pad, cxg_pad, gix_pad, coord_pad)


def _k3_body(pm_ref, s_ref, h0_ref, h1_ref, latent_ref, rw_ref, rb_ref,
             ls_ref, out_ref):
    pm = pm_ref[...]
    sv = s_ref[...]
    r0 = lax.broadcasted_iota(jnp.int32, pm.shape, 0)
    c0 = lax.broadcasted_iota(jnp.int32, pm.shape, 1)
    maskc = (r0 * 128 + c0) < NC
    mix = jnp.sum(jnp.where(maskc, pm - jnp.log(jnp.where(maskc, sv, 1.0)),
                            0.0))
    mix = mix + jnp.float32(NC) * jnp.log(jnp.float32(C))

    fc = (h0_ref[...] + h1_ref[...]).astype(jnp.float32)
    rho = lax.dot_general(latent_ref[...], rw_ref[...],
                          (((1,), (1,)), ((), ())),
                          preferred_element_type=jnp.float32,
                          precision=lax.Precision.HIGHEST)
    fe = rb_ref[...] * jnp.exp(rho) * ls_ref[...]
    x = fc + 1.0
    z = x + 7.0
    prod = (x * (x + 1.0) * (x + 2.0) * (x + 3.0) * (x + 4.0) * (x + 5.0)
            * (x + 6.0))
    zi = 1.0 / z
    zi2 = zi * zi
    lg = ((z - 0.5) * jnp.log(z) - z + jnp.float32(0.9189385332046727)
          + zi * (jnp.float32(1.0 / 12.0)
                  - zi2 * (jnp.float32(1.0 / 360.0)
                           - zi2 * jnp.float32(1.0 / 1260.0)))
          - jnp.log(prod))
    lfc = fc * jnp.log(fe) - fe - lg
    out_ref[0, 0] = -(mix + jnp.sum(lfc))


def _k3(pm2, s2, h0, h1, latent, rwoi, rb_row, ls_col):
    return pl.pallas_call(
        _k3_body,
        out_shape=jax.ShapeDtypeStruct((1, 1), jnp.float32),
        out_specs=pl.BlockSpec(memory_space=pltpu.SMEM),
    )(pm2, s2, h0, h1, latent, rwoi, rb_row, ls_col)


def kernel(latent, genes_oi, cells_oi, cut_coordinates, cut_local_cellxgene_ix,
           cut_local_gene_ix, local_cellxgene_ix, n_cells, n_genes,
           logit_weight, rho_weight, bin_logit_baseline, rho_bias, libsize):
    genes_oi = genes_oi.astype(jnp.int32)
    cells_oi = cells_oi.astype(jnp.int32)
    cxg = cut_local_cellxgene_ix.astype(jnp.int32)
    gix = cut_local_gene_ix.astype(jnp.int32)
    frag = local_cellxgene_ix.astype(jnp.int32)

    md_flat = _k1(latent, genes_oi, logit_weight)

    genes_pad = jnp.pad(genes_oi, (0, 512 - G))
    cxg_pad = jnp.pad(cxg, (0, NCPAD - NC))
    gix_pad = jnp.pad(gix, (0, NCPAD - NC))
    coord_pad = jnp.pad(cut_coordinates, (0, NCPAD - NC))
    frag_pad = jnp.pad(frag, (0, NFPAD - NF), constant_values=BG)
    rb_pad = jnp.pad(rho_bias, (0, 5120 - NGT))
    rw_pad = jnp.pad(rho_weight, ((0, 0), (0, C - L)))

    hist, rboi, lsoi, rwoi = _k2a(genes_pad, frag_pad, rb_pad, libsize,
                                  cells_oi, rw_pad)
    pm, sv = _k2b(md_flat, bin_logit_baseline, genes_pad, cxg_pad, gix_pad,
                  coord_pad)

    pm2 = pm.reshape(NCPAD // 128, 128)
    s2 = sv.reshape(NCPAD // 128, 128)
    h0 = hist[0, :BG].reshape(B, G)
    h1 = hist[1, :BG].reshape(B, G)
    rb_row = rboi[:G].reshape(1, G)
    ls_col = lsoi.reshape(B, 1)

    out = _k3(pm2, s2, h0, h1, latent, rwoi[:G, :L], rb_row, ls_col)
    scale = (jnp.asarray(n_cells, jnp.float32) * jnp.asarray(n_genes, jnp.float32)
             / jnp.float32(BG))
    return out[0, 0] * scale


# speedup vs baseline: 6.3358x; 1.1677x over previous
import functools

import jax
import jax.numpy as jnp
from jax import lax
from jax.experimental import pallas as pl
from jax.experimental.pallas import tpu as pltpu
from jax.experimental.pallas import tpu_sc as plsc

B = 256
G = 500
L = 32
C = 128
NGT = 5000
NC = 200000
NF = 400000
NW = 32
CH = 128
NC_CHUNKS = 1568
NCPAD = NC_CHUNKS * CH
NF_CHUNKS = 3136
NFPAD = NF_CHUNKS * CH
H = 131072
BG = B * G


GPS = 4


def _k1_body(genes_ref, latent_ref, lw_hbm, md_ref, lw_scr, sem):
    j = pl.program_id(0)
    nsteps = pl.num_programs(0)
    latent = latent_ref[...]

    def fire(jj, slot):
        for k in range(GPS):
            pltpu.make_async_copy(lw_hbm.at[genes_ref[jj * GPS + k]],
                                  lw_scr.at[slot, k], sem).start()

    @pl.when(j == 0)
    def _prime():
        fire(0, 0)

    @pl.when(j < nsteps - 1)
    def _next():
        fire(j + 1, lax.rem(j + 1, 2))

    slot = lax.rem(j, 2)
    for k in range(GPS):
        pltpu.make_async_copy(lw_hbm.at[genes_ref[j * GPS + k]],
                              lw_scr.at[slot, k], sem).wait()
        md_ref[pl.ds(k * B, B), :] = jnp.dot(
            latent, lw_scr[slot, k], preferred_element_type=jnp.float32)


def _k1(latent, genes_oi, logit_weight):
    grid_spec = pltpu.PrefetchScalarGridSpec(
        num_scalar_prefetch=1,
        grid=(G // GPS,),
        in_specs=[
            pl.BlockSpec((B, L), lambda g, gref: (0, 0)),
            pl.BlockSpec(memory_space=pl.ANY),
        ],
        out_specs=[
            pl.BlockSpec((GPS * B, C), lambda g, gref: (g, 0)),
        ],
        scratch_shapes=[
            pltpu.VMEM((2, GPS, L, C), jnp.float32),
            pltpu.SemaphoreType.DMA,
        ],
    )
    return pl.pallas_call(
        _k1_body,
        grid_spec=grid_spec,
        out_shape=[
            jax.ShapeDtypeStruct((G * B, C), jnp.float32),
        ],
    )(genes_oi, latent, logit_weight)[0]


def _k2a_body(genes_hbm, frag_hbm, rb_hbm, ls_hbm, cells_hbm, rw_hbm,
              hist_hbm, rboi_hbm, lsoi_hbm, rwoi_hbm,
              genes_v, frag_a, frag_b, ones_v, zeros_v,
              rb_v, ls_v, cells_v, rboi_v, lsoi_v, gidx_v, rwoi_v,
              hist_sh, sem_f_a, sem_f_b, sem1):
    c = lax.axis_index("c")
    s = lax.axis_index("s")
    wid = s * 2 + c
    NJF = NF_CHUNKS // NW
    fbase = wid * (NJF * CH)

    pltpu.sync_copy(genes_hbm, genes_v)

    @pl.when(s == 0)
    def _zero_hist():
        def zv(i, carry):
            zeros_v[pl.ds(i * 16, 16)] = jnp.zeros((16,), jnp.int32)
            return carry
        lax.fori_loop(0, 64, zv, 0)

        def zh(k, carry):
            pltpu.sync_copy(zeros_v, hist_sh.at[pl.ds(k * 1024, 1024)])
            return carry
        lax.fori_loop(0, H // 1024, zh, 0)

    for k in range(8):
        ones_v[pl.ds(k * 16, 16)] = jnp.full((16,), 1, jnp.int32)

    plsc.subcore_barrier()

    def ffire(j, fb, semf):
        toff = pl.multiple_of(fbase + j * CH, CH)
        pltpu.async_copy(frag_hbm.at[pl.ds(toff, CH)], fb, semf)

    ffire(0, frag_a, sem_f_a)

    def frag_step(j, carry):
        even = lax.rem(j, 2) == 0

        @pl.when(jnp.logical_and(even, j < NJF - 1))
        def _fb():
            ffire(j + 1, frag_b, sem_f_b)

        @pl.when(jnp.logical_and(jnp.logical_not(even), j < NJF - 1))
        def _fa():
            ffire(j + 1, frag_a, sem_f_a)

        @pl.when(even)
        def _sa():
            pltpu.make_async_copy(frag_hbm.at[pl.ds(0, CH)], frag_a,
                                  sem_f_a).wait()
            pltpu.sync_copy(ones_v, hist_sh.at[frag_a], add=True)

        @pl.when(jnp.logical_not(even))
        def _sb():
            pltpu.make_async_copy(frag_hbm.at[pl.ds(0, CH)], frag_b,
                                  sem_f_b).wait()
            pltpu.sync_copy(ones_v, hist_sh.at[frag_b], add=True)
        return carry
    lax.fori_loop(0, NJF, frag_step, 0)

    @pl.when(jnp.logical_and(c == 0, s == 0))
    def _small_gathers():
        pltpu.sync_copy(rb_hbm, rb_v)
        pltpu.sync_copy(ls_hbm, ls_v)
        pltpu.sync_copy(cells_hbm, cells_v)
        for k in range(512 // 16):
            sl = pl.ds(k * 16, 16)
            rboi_v[sl] = plsc.load_gather(rb_v, [genes_v[sl]])
        for k in range(256 // 16):
            sl = pl.ds(k * 16, 16)
            lsoi_v[sl] = plsc.load_gather(ls_v, [cells_v[sl]])
        pltpu.sync_copy(rboi_v, rboi_hbm)
        pltpu.sync_copy(lsoi_v, lsoi_hbm)
        for k in range(512 // CH):
            pltpu.sync_copy(genes_hbm.at[pl.ds(k * CH, CH)], gidx_v)
            pltpu.async_copy(rw_hbm.at[gidx_v], rwoi_v, sem1).wait()
            pltpu.sync_copy(rwoi_v, rwoi_hbm.at[pl.ds(k * CH, CH)])

    plsc.subcore_barrier()

    @pl.when(s == 0)
    def _export_hist():
        pltpu.sync_copy(hist_sh, hist_hbm.at[c])


def _k2a(genes_pad, frag_pad, rb_pad, ls, cells, rw):
    kfn = functools.partial(
        pl.kernel,
        out_type=[
            jax.ShapeDtypeStruct((2, H), jnp.int32),
            jax.ShapeDtypeStruct((512,), jnp.float32),
            jax.ShapeDtypeStruct((256,), jnp.float32),
            jax.ShapeDtypeStruct((512, C), jnp.float32),
        ],
        mesh=plsc.VectorSubcoreMesh(core_axis_name="c", subcore_axis_name="s"),
        compiler_params=pltpu.CompilerParams(needs_layout_passes=False),
        scratch_types=[
            pltpu.VMEM((512,), jnp.int32),
            pltpu.VMEM((CH,), jnp.int32),
            pltpu.VMEM((CH,), jnp.int32),
            pltpu.VMEM((CH,), jnp.int32),
            pltpu.VMEM((1024,), jnp.int32),
            pltpu.VMEM((5120,), jnp.float32),
            pltpu.VMEM((10000,), jnp.float32),
            pltpu.VMEM((256,), jnp.int32),
            pltpu.VMEM((512,), jnp.float32),
            pltpu.VMEM((256,), jnp.float32),
            pltpu.VMEM((CH,), jnp.int32),
            pltpu.VMEM((CH, C), jnp.float32),
            pltpu.VMEM_SHARED((H,), jnp.int32),
            pltpu.SemaphoreType.DMA,
            pltpu.SemaphoreType.DMA,
            pltpu.SemaphoreType.DMA,
        ],
    )
    return kfn(_k2a_body)(genes_pad, frag_pad, rb_pad, ls, cells, rw)


def _k2b_body(md_hbm, bl_hbm, genes_hbm, cxg_hbm, gix_hbm, coord_hbm,
              pm_hbm, s_hbm,
              genes_v, idx_all, gix_all, coord_all, pm_v, s_v,
              idx_a, gg_a, bins_a, md_a, bl_a,
              idx_b, gg_b, bins_b, md_b, bl_b,
              sem_md_a, sem_bl_a, sem_md_b, sem_bl_b):
    c = lax.axis_index("c")
    s = lax.axis_index("s")
    wid = s * 2 + c
    NJ = NC_CHUNKS // NW
    cbase = wid * (NJ * CH)

    pltpu.sync_copy(genes_hbm, genes_v)
    pltpu.sync_copy(cxg_hbm.at[pl.ds(cbase, NJ * CH)], idx_all)
    pltpu.sync_copy(gix_hbm.at[pl.ds(cbase, NJ * CH)], gix_all)
    pltpu.sync_copy(coord_hbm.at[pl.ds(cbase, NJ * CH)], coord_all)

    lane = lax.iota(jnp.int32, 16)
    lane0 = lane == 0

    def fire(j, idxb, ggb, binsb, mdb, blb, sem_md, sem_bl):
        for k in range(CH // 16):
            sl = pl.ds(k * 16, 16)
            gl = pl.ds(j * CH + k * 16, 16)
            ix = idx_all[gl]
            idxb[sl] = lax.rem(ix, jnp.int32(G)) * B + lax.div(ix, jnp.int32(G))
            ggb[sl] = plsc.load_gather(genes_v, [gix_all[gl]])
            b = (coord_all[gl] * jnp.float32(C)).astype(jnp.int32)
            binsb[sl] = jnp.clip(b, 0, C - 1)
        pltpu.async_copy(md_hbm.at[idxb], mdb, sem_md)
        pltpu.async_copy(bl_hbm.at[ggb], blb, sem_bl)

    def compute(j, idxb, ggb, binsb, mdb, blb, sem_md, sem_bl):
        pltpu.make_async_copy(md_hbm.at[idxb], mdb, sem_md).wait()
        pltpu.make_async_copy(bl_hbm.at[ggb], blb, sem_bl).wait()
        toff = pl.multiple_of(cbase + j * CH, CH)
        for k in range(CH // 16):
            sl = pl.ds(k * 16, 16)
            rows = lane + jnp.int32(k * 16)
            cols = binsb[sl]
            pm_v[sl] = (plsc.load_gather(mdb, [rows, cols])
                        + plsc.load_gather(blb, [rows, cols]))

        def cut_body(i2, carry2):
            accs = []
            for u in range(2):
                i = i2 * 2 + u
                acc = jnp.zeros((16,), jnp.float32)
                for k in range(C // 16):
                    sl = pl.ds(k * 16, 16)
                    acc = acc + jnp.exp(mdb[i, sl] + blb[i, sl])
                accs.append(acc)
            for u in range(2):
                i = i2 * 2 + u
                ii = jnp.full((16,), i, jnp.int32)
                plsc.store_scatter(s_v, [ii],
                                   jnp.sum(accs[u]) + jnp.zeros((16,),
                                                                jnp.float32),
                                   mask=lane0)
            return carry2
        lax.fori_loop(0, CH // 2, cut_body, 0)

        pltpu.sync_copy(pm_v, pm_hbm.at[pl.ds(toff, CH)])
        pltpu.sync_copy(s_v, s_hbm.at[pl.ds(toff, CH)])

    bufs_a = (idx_a, gg_a, bins_a, md_a, bl_a, sem_md_a, sem_bl_a)
    bufs_b = (idx_b, gg_b, bins_b, md_b, bl_b, sem_md_b, sem_bl_b)
    fire(0, *bufs_a)

    def cut_step(j, carry):
        even = lax.rem(j, 2) == 0

        @pl.when(jnp.logical_and(even, j < NJ - 1))
        def _fb():
            fire(j + 1, *bufs_b)

        @pl.when(jnp.logical_and(jnp.logical_not(even), j < NJ - 1))
        def _fa():
            fire(j + 1, *bufs_a)

        @pl.when(even)
        def _ca():
            compute(j, *bufs_a)

        @pl.when(jnp.logical_not(even))
        def _cb():
            compute(j, *bufs_b)
        return carry
    lax.fori_loop(0, NJ, cut_step, 0)


def _k2b(md_flat, bl, genes_pad, cxg_pad, gix_pad, coord_pad):
    kfn = functools.partial(
        pl.kernel,
        out_type=[
            jax.ShapeDtypeStruct((NCPAD,), jnp.float32),
            jax.ShapeDtypeStruct((NCPAD,), jnp.float32),
        ],
        mesh=plsc.VectorSubcoreMesh(core_axis_name="c", subcore_axis_name="s"),
        compiler_params=pltpu.CompilerParams(needs_layout_passes=False),
        scratch_types=[
            pltpu.VMEM((512,), jnp.int32),
            pltpu.VMEM((NCPAD // NW,), jnp.int32),
            pltpu.VMEM((NCPAD // NW,), jnp.int32),
            pltpu.VMEM((NCPAD // NW,), jnp.float32),
            pltpu.VMEM((CH,), jnp.float32),
            pltpu.VMEM((CH,), jnp.float32),
            pltpu.VMEM((CH,), jnp.int32),
            pltpu.VMEM((CH,), jnp.int32),
            pltpu.VMEM((CH,), jnp.int32),
            pltpu.VMEM((CH, C), jnp.float32),
            pltpu.VMEM((CH, C), jnp.float32),
            pltpu.VMEM((CH,), jnp.int32),
            pltpu.VMEM((CH,), jnp.int32),
            pltpu.VMEM((CH,), jnp.int32),
            pltpu.VMEM((CH, C), jnp.float32),
            pltpu.VMEM((CH, C), jnp.float32),
            pltpu.SemaphoreType.DMA,
            pltpu.SemaphoreType.DMA,
            pltpu.SemaphoreType.DMA,
            pltpu.SemaphoreType.DMA,
        ],
    )
    return kfn(_k2b_body)(md_flat, bl, genes_pad, cxg_pad, gix_pad, coord_pad)


def _k3_body(pm_ref, s_ref, h0_ref, h1_ref, latent_ref, rw_ref, rb_ref,
             ls_ref, out_ref):
    pm = pm_ref[...]
    sv = s_ref[...]
    r0 = lax.broadcasted_iota(jnp.int32, pm.shape, 0)
    c0 = lax.broadcasted_iota(jnp.int32, pm.shape, 1)
    maskc = (r0 * 128 + c0) < NC
    mix = jnp.sum(jnp.where(maskc, pm - jnp.log(jnp.where(maskc, sv, 1.0)),
                            0.0))
    mix = mix + jnp.float32(NC) * jnp.log(jnp.float32(C))

    fc = (h0_ref[...] + h1_ref[...]).astype(jnp.float32)
    rho = lax.dot_general(latent_ref[...], rw_ref[...],
                          (((1,), (1,)), ((), ())),
                          preferred_element_type=jnp.float32,
                          precision=lax.Precision.HIGHEST)
    fe = rb_ref[...] * jnp.exp(rho) * ls_ref[...]
    x = fc + 1.0
    z = x + 7.0
    prod = (x * (x + 1.0) * (x + 2.0) * (x + 3.0) * (x + 4.0) * (x + 5.0)
            * (x + 6.0))
    zi = 1.0 / z
    zi2 = zi * zi
    lg = ((z - 0.5) * jnp.log(z) - z + jnp.float32(0.9189385332046727)
          + zi * (jnp.float32(1.0 / 12.0)
                  - zi2 * (jnp.float32(1.0 / 360.0)
                           - zi2 * jnp.float32(1.0 / 1260.0)))
          - jnp.log(prod))
    lfc = fc * jnp.log(fe) - fe - lg
    out_ref[0, 0] = -(mix + jnp.sum(lfc))


def _k3(pm2, s2, h0, h1, latent, rwoi, rb_row, ls_col):
    return pl.pallas_call(
        _k3_body,
        out_shape=jax.ShapeDtypeStruct((1, 1), jnp.float32),
        out_specs=pl.BlockSpec(memory_space=pltpu.SMEM),
    )(pm2, s2, h0, h1, latent, rwoi, rb_row, ls_col)


def kernel(latent, genes_oi, cells_oi, cut_coordinates, cut_local_cellxgene_ix,
           cut_local_gene_ix, local_cellxgene_ix, n_cells, n_genes,
           logit_weight, rho_weight, bin_logit_baseline, rho_bias, libsize):
    genes_oi = genes_oi.astype(jnp.int32)
    cells_oi = cells_oi.astype(jnp.int32)
    cxg = cut_local_cellxgene_ix.astype(jnp.int32)
    gix = cut_local_gene_ix.astype(jnp.int32)
    frag = local_cellxgene_ix.astype(jnp.int32)

    md_flat = _k1(latent, genes_oi, logit_weight)

    genes_pad = jnp.pad(genes_oi, (0, 512 - G))
    cxg_pad = jnp.pad(cxg, (0, NCPAD - NC))
    gix_pad = jnp.pad(gix, (0, NCPAD - NC))
    coord_pad = jnp.pad(cut_coordinates, (0, NCPAD - NC))
    frag_pad = jnp.pad(frag, (0, NFPAD - NF), constant_values=BG)
    rb_pad = jnp.pad(rho_bias, (0, 5120 - NGT))
    rw_pad = jnp.pad(rho_weight, ((0, 0), (0, C - L)))

    hist, rboi, lsoi, rwoi = _k2a(genes_pad, frag_pad, rb_pad, libsize,
                                  cells_oi, rw_pad)
    pm, sv = _k2b(md_flat, bin_logit_baseline, genes_pad, cxg_pad, gix_pad,
                  coord_pad)

    pm2 = pm.reshape(NCPAD // 128, 128)
    s2 = sv.reshape(NCPAD // 128, 128)
    h0 = hist[0, :BG].reshape(B, G)
    h1 = hist[1, :BG].reshape(B, G)
    rb_row = rboi[:G].reshape(1, G)
    ls_col = lsoi.reshape(B, 1)

    out = _k3(pm2, s2, h0, h1, latent, rwoi[:G, :L], rb_row, ls_col)
    scale = (jnp.asarray(n_cells, jnp.float32) * jnp.asarray(n_genes, jnp.float32)
             / jnp.float32(BG))
    return out[0, 0] * scale


# speedup vs baseline: 7.0242x; 1.1087x over previous
import functools

import jax
import jax.numpy as jnp
from jax import lax
from jax.experimental import pallas as pl
from jax.experimental.pallas import tpu as pltpu
from jax.experimental.pallas import tpu_sc as plsc

B = 256
G = 500
L = 32
C = 128
NGT = 5000
NC = 200000
NF = 400000
NW = 32
CH = 128
NC_CHUNKS = 1568
NCPAD = NC_CHUNKS * CH
NF_CHUNKS = 3136
NFPAD = NF_CHUNKS * CH
H = 131072
BG = B * G


GPS = 4


def _k1_body(genes_ref, latent_ref, lw_hbm, md_ref, lw_scr, sem):
    j = pl.program_id(0)
    nsteps = pl.num_programs(0)
    latent = latent_ref[...]

    def fire(jj, slot):
        for k in range(GPS):
            pltpu.make_async_copy(lw_hbm.at[genes_ref[jj * GPS + k]],
                                  lw_scr.at[slot, k], sem).start()

    @pl.when(j == 0)
    def _prime():
        fire(0, 0)

    @pl.when(j < nsteps - 1)
    def _next():
        fire(j + 1, lax.rem(j + 1, 2))

    slot = lax.rem(j, 2)
    for k in range(GPS):
        pltpu.make_async_copy(lw_hbm.at[genes_ref[j * GPS + k]],
                              lw_scr.at[slot, k], sem).wait()
        md_ref[pl.ds(k * B, B), :] = jnp.dot(
            latent, lw_scr[slot, k], preferred_element_type=jnp.float32)


def _k1(latent, genes_oi, logit_weight):
    grid_spec = pltpu.PrefetchScalarGridSpec(
        num_scalar_prefetch=1,
        grid=(G // GPS,),
        in_specs=[
            pl.BlockSpec((B, L), lambda g, gref: (0, 0)),
            pl.BlockSpec(memory_space=pl.ANY),
        ],
        out_specs=[
            pl.BlockSpec((GPS * B, C), lambda g, gref: (g, 0)),
        ],
        scratch_shapes=[
            pltpu.VMEM((2, GPS, L, C), jnp.float32),
            pltpu.SemaphoreType.DMA,
        ],
    )
    return pl.pallas_call(
        _k1_body,
        grid_spec=grid_spec,
        out_shape=[
            jax.ShapeDtypeStruct((G * B, C), jnp.float32),
        ],
    )(genes_oi, latent, logit_weight)[0]


def _k2a_body(genes_hbm, frag_hbm, rb_hbm, ls_hbm, cells_hbm, rw_hbm,
              hist_hbm, rboi_hbm, lsoi_hbm, rwoi_hbm,
              genes_v, frag_a, frag_b, ones_v, zeros_v,
              rb_v, ls_v, cells_v, rboi_v, lsoi_v, gidx_v, rwoi_v,
              hist_sh, sem_f_a, sem_f_b, sem1):
    c = lax.axis_index("c")
    s = lax.axis_index("s")
    wid = s * 2 + c
    NJF = NF_CHUNKS // NW
    fbase = wid * (NJF * CH)

    pltpu.sync_copy(genes_hbm, genes_v)

    @pl.when(s == 0)
    def _zero_hist():
        def zv(i, carry):
            zeros_v[pl.ds(i * 16, 16)] = jnp.zeros((16,), jnp.int32)
            return carry
        lax.fori_loop(0, 64, zv, 0)

        def zh(k, carry):
            pltpu.sync_copy(zeros_v, hist_sh.at[pl.ds(k * 1024, 1024)])
            return carry
        lax.fori_loop(0, H // 1024, zh, 0)

    for k in range(8):
        ones_v[pl.ds(k * 16, 16)] = jnp.full((16,), 1, jnp.int32)

    plsc.subcore_barrier()

    def ffire(j, fb, semf):
        toff = pl.multiple_of(fbase + j * CH, CH)
        pltpu.async_copy(frag_hbm.at[pl.ds(toff, CH)], fb, semf)

    ffire(0, frag_a, sem_f_a)

    def frag_step(j, carry):
        even = lax.rem(j, 2) == 0

        @pl.when(jnp.logical_and(even, j < NJF - 1))
        def _fb():
            ffire(j + 1, frag_b, sem_f_b)

        @pl.when(jnp.logical_and(jnp.logical_not(even), j < NJF - 1))
        def _fa():
            ffire(j + 1, frag_a, sem_f_a)

        @pl.when(even)
        def _sa():
            pltpu.make_async_copy(frag_hbm.at[pl.ds(0, CH)], frag_a,
                                  sem_f_a).wait()
            pltpu.sync_copy(ones_v, hist_sh.at[frag_a], add=True)

        @pl.when(jnp.logical_not(even))
        def _sb():
            pltpu.make_async_copy(frag_hbm.at[pl.ds(0, CH)], frag_b,
                                  sem_f_b).wait()
            pltpu.sync_copy(ones_v, hist_sh.at[frag_b], add=True)
        return carry
    lax.fori_loop(0, NJF, frag_step, 0)

    @pl.when(jnp.logical_and(c == 0, s == 0))
    def _small_gathers():
        pltpu.sync_copy(rb_hbm, rb_v)
        pltpu.sync_copy(ls_hbm, ls_v)
        pltpu.sync_copy(cells_hbm, cells_v)
        for k in range(512 // 16):
            sl = pl.ds(k * 16, 16)
            rboi_v[sl] = plsc.load_gather(rb_v, [genes_v[sl]])
        for k in range(256 // 16):
            sl = pl.ds(k * 16, 16)
            lsoi_v[sl] = plsc.load_gather(ls_v, [cells_v[sl]])
        pltpu.sync_copy(rboi_v, rboi_hbm)
        pltpu.sync_copy(lsoi_v, lsoi_hbm)
        for k in range(512 // CH):
            pltpu.sync_copy(genes_hbm.at[pl.ds(k * CH, CH)], gidx_v)
            pltpu.async_copy(rw_hbm.at[gidx_v], rwoi_v, sem1).wait()
            pltpu.sync_copy(rwoi_v, rwoi_hbm.at[pl.ds(k * CH, CH)])

    plsc.subcore_barrier()

    @pl.when(s == 0)
    def _export_hist():
        pltpu.sync_copy(hist_sh, hist_hbm.at[c])


def _k2a(genes_pad, frag_pad, rb_pad, ls, cells, rw):
    kfn = functools.partial(
        pl.kernel,
        out_type=[
            jax.ShapeDtypeStruct((2, H), jnp.int32),
            jax.ShapeDtypeStruct((512,), jnp.float32),
            jax.ShapeDtypeStruct((256,), jnp.float32),
            jax.ShapeDtypeStruct((512, C), jnp.float32),
        ],
        mesh=plsc.VectorSubcoreMesh(core_axis_name="c", subcore_axis_name="s"),
        compiler_params=pltpu.CompilerParams(needs_layout_passes=False),
        scratch_types=[
            pltpu.VMEM((512,), jnp.int32),
            pltpu.VMEM((CH,), jnp.int32),
            pltpu.VMEM((CH,), jnp.int32),
            pltpu.VMEM((CH,), jnp.int32),
            pltpu.VMEM((1024,), jnp.int32),
            pltpu.VMEM((5120,), jnp.float32),
            pltpu.VMEM((10000,), jnp.float32),
            pltpu.VMEM((256,), jnp.int32),
            pltpu.VMEM((512,), jnp.float32),
            pltpu.VMEM((256,), jnp.float32),
            pltpu.VMEM((CH,), jnp.int32),
            pltpu.VMEM((CH, C), jnp.float32),
            pltpu.VMEM_SHARED((H,), jnp.int32),
            pltpu.SemaphoreType.DMA,
            pltpu.SemaphoreType.DMA,
            pltpu.SemaphoreType.DMA,
        ],
    )
    return kfn(_k2a_body)(genes_pad, frag_pad, rb_pad, ls, cells, rw)


def _k2b_body(md_hbm, bl_hbm, genes_hbm, cxg_hbm, gix_hbm, coord_hbm,
              pm_hbm, s_hbm,
              genes_v, idx_all, gix_all, coord_all, pm_v, s_v,
              idx_a, gg_a, bins_a, md_a, bl_a,
              idx_b, gg_b, bins_b, md_b, bl_b,
              sem_md_a, sem_bl_a, sem_md_b, sem_bl_b):
    c = lax.axis_index("c")
    s = lax.axis_index("s")
    wid = s * 2 + c
    NJ = NC_CHUNKS // NW
    cbase = wid * (NJ * CH)

    pltpu.sync_copy(genes_hbm, genes_v)
    pltpu.sync_copy(cxg_hbm.at[pl.ds(cbase, NJ * CH)], idx_all)
    pltpu.sync_copy(gix_hbm.at[pl.ds(cbase, NJ * CH)], gix_all)
    pltpu.sync_copy(coord_hbm.at[pl.ds(cbase, NJ * CH)], coord_all)

    lane = lax.iota(jnp.int32, 16)
    lane0 = lane == 0

    def fire(j, idxb, ggb, binsb, mdb, blb, sem_md, sem_bl):
        for k in range(CH // 16):
            sl = pl.ds(k * 16, 16)
            gl = pl.ds(j * CH + k * 16, 16)
            ix = idx_all[gl]
            idxb[sl] = lax.rem(ix, jnp.int32(G)) * B + lax.div(ix, jnp.int32(G))
            ggb[sl] = plsc.load_gather(genes_v, [gix_all[gl]])
            b = (coord_all[gl] * jnp.float32(C)).astype(jnp.int32)
            binsb[sl] = jnp.clip(b, 0, C - 1)
        pltpu.async_copy(md_hbm.at[idxb], mdb, sem_md)
        pltpu.async_copy(bl_hbm.at[ggb], blb, sem_bl)

    def compute(j, idxb, ggb, binsb, mdb, blb, sem_md, sem_bl):
        pltpu.make_async_copy(md_hbm.at[idxb], mdb, sem_md).wait()
        pltpu.make_async_copy(bl_hbm.at[ggb], blb, sem_bl).wait()
        toff = pl.multiple_of(cbase + j * CH, CH)
        for k in range(CH // 16):
            sl = pl.ds(k * 16, 16)
            rows = lane + jnp.int32(k * 16)
            cols = binsb[sl]
            pm_v[sl] = (plsc.load_gather(mdb, [rows, cols])
                        + plsc.load_gather(blb, [rows, cols]))

        def cut_body(i2, carry2):
            accs = []
            for u in range(4):
                i = i2 * 4 + u
                acc = jnp.zeros((16,), jnp.float32)
                for k in range(C // 16):
                    sl = pl.ds(k * 16, 16)
                    acc = acc + jnp.exp(mdb[i, sl] + blb[i, sl])
                accs.append(acc)
            for u in range(4):
                i = i2 * 4 + u
                ii = jnp.full((16,), i, jnp.int32)
                plsc.store_scatter(s_v, [ii],
                                   jnp.sum(accs[u]) + jnp.zeros((16,),
                                                                jnp.float32),
                                   mask=lane0)
            return carry2
        lax.fori_loop(0, CH // 4, cut_body, 0)

        pltpu.sync_copy(pm_v, pm_hbm.at[pl.ds(toff, CH)])
        pltpu.sync_copy(s_v, s_hbm.at[pl.ds(toff, CH)])

    bufs_a = (idx_a, gg_a, bins_a, md_a, bl_a, sem_md_a, sem_bl_a)
    bufs_b = (idx_b, gg_b, bins_b, md_b, bl_b, sem_md_b, sem_bl_b)
    fire(0, *bufs_a)

    def cut_step(j, carry):
        even = lax.rem(j, 2) == 0

        @pl.when(jnp.logical_and(even, j < NJ - 1))
        def _fb():
            fire(j + 1, *bufs_b)

        @pl.when(jnp.logical_and(jnp.logical_not(even), j < NJ - 1))
        def _fa():
            fire(j + 1, *bufs_a)

        @pl.when(even)
        def _ca():
            compute(j, *bufs_a)

        @pl.when(jnp.logical_not(even))
        def _cb():
            compute(j, *bufs_b)
        return carry
    lax.fori_loop(0, NJ, cut_step, 0)


def _k2b(md_flat, bl, genes_pad, cxg_pad, gix_pad, coord_pad):
    kfn = functools.partial(
        pl.kernel,
        out_type=[
            jax.ShapeDtypeStruct((NCPAD,), jnp.float32),
            jax.ShapeDtypeStruct((NCPAD,), jnp.float32),
        ],
        mesh=plsc.VectorSubcoreMesh(core_axis_name="c", subcore_axis_name="s"),
        compiler_params=pltpu.CompilerParams(needs_layout_passes=False),
        scratch_types=[
            pltpu.VMEM((512,), jnp.int32),
            pltpu.VMEM((NCPAD // NW,), jnp.int32),
            pltpu.VMEM((NCPAD // NW,), jnp.int32),
            pltpu.VMEM((NCPAD // NW,), jnp.float32),
            pltpu.VMEM((CH,), jnp.float32),
            pltpu.VMEM((CH,), jnp.float32),
            pltpu.VMEM((CH,), jnp.int32),
            pltpu.VMEM((CH,), jnp.int32),
            pltpu.VMEM((CH,), jnp.int32),
            pltpu.VMEM((CH, C), jnp.float32),
            pltpu.VMEM((CH, C), jnp.float32),
            pltpu.VMEM((CH,), jnp.int32),
            pltpu.VMEM((CH,), jnp.int32),
            pltpu.VMEM((CH,), jnp.int32),
            pltpu.VMEM((CH, C), jnp.float32),
            pltpu.VMEM((CH, C), jnp.float32),
            pltpu.SemaphoreType.DMA,
            pltpu.SemaphoreType.DMA,
            pltpu.SemaphoreType.DMA,
            pltpu.SemaphoreType.DMA,
        ],
    )
    return kfn(_k2b_body)(md_flat, bl, genes_pad, cxg_pad, gix_pad, coord_pad)


def _k3_body(pm_ref, s_ref, h0_ref, h1_ref, latent_ref, rw_ref, rb_ref,
             ls_ref, out_ref):
    pm = pm_ref[...]
    sv = s_ref[...]
    r0 = lax.broadcasted_iota(jnp.int32, pm.shape, 0)
    c0 = lax.broadcasted_iota(jnp.int32, pm.shape, 1)
    maskc = (r0 * 128 + c0) < NC
    mix = jnp.sum(jnp.where(maskc, pm - jnp.log(jnp.where(maskc, sv, 1.0)),
                            0.0))
    mix = mix + jnp.float32(NC) * jnp.log(jnp.float32(C))

    fc = (h0_ref[...] + h1_ref[...]).astype(jnp.float32)
    rho = lax.dot_general(latent_ref[...], rw_ref[...],
                          (((1,), (1,)), ((), ())),
                          preferred_element_type=jnp.float32,
                          precision=lax.Precision.HIGHEST)
    fe = rb_ref[...] * jnp.exp(rho) * ls_ref[...]
    x = fc + 1.0
    z = x + 7.0
    prod = (x * (x + 1.0) * (x + 2.0) * (x + 3.0) * (x + 4.0) * (x + 5.0)
            * (x + 6.0))
    zi = 1.0 / z
    zi2 = zi * zi
    lg = ((z - 0.5) * jnp.log(z) - z + jnp.float32(0.9189385332046727)
          + zi * (jnp.float32(1.0 / 12.0)
                  - zi2 * (jnp.float32(1.0 / 360.0)
                           - zi2 * jnp.float32(1.0 / 1260.0)))
          - jnp.log(prod))
    lfc = fc * jnp.log(fe) - fe - lg
    out_ref[0, 0] = -(mix + jnp.sum(lfc))


def _k3(pm2, s2, h0, h1, latent, rwoi, rb_row, ls_col):
    return pl.pallas_call(
        _k3_body,
        out_shape=jax.ShapeDtypeStruct((1, 1), jnp.float32),
        out_specs=pl.BlockSpec(memory_space=pltpu.SMEM),
    )(pm2, s2, h0, h1, latent, rwoi, rb_row, ls_col)


def kernel(latent, genes_oi, cells_oi, cut_coordinates, cut_local_cellxgene_ix,
           cut_local_gene_ix, local_cellxgene_ix, n_cells, n_genes,
           logit_weight, rho_weight, bin_logit_baseline, rho_bias, libsize):
    genes_oi = genes_oi.astype(jnp.int32)
    cells_oi = cells_oi.astype(jnp.int32)
    cxg = cut_local_cellxgene_ix.astype(jnp.int32)
    gix = cut_local_gene_ix.astype(jnp.int32)
    frag = local_cellxgene_ix.astype(jnp.int32)

    md_flat = _k1(latent, genes_oi, logit_weight)

    genes_pad = jnp.pad(genes_oi, (0, 512 - G))
    cxg_pad = jnp.pad(cxg, (0, NCPAD - NC))
    gix_pad = jnp.pad(gix, (0, NCPAD - NC))
    coord_pad = jnp.pad(cut_coordinates, (0, NCPAD - NC))
    frag_pad = jnp.pad(frag, (0, NFPAD - NF), constant_values=BG)
    rb_pad = jnp.pad(rho_bias, (0, 5120 - NGT))
    rw_pad = jnp.pad(rho_weight, ((0, 0), (0, C - L)))

    hist, rboi, lsoi, rwoi = _k2a(genes_pad, frag_pad, rb_pad, libsize,
                                  cells_oi, rw_pad)
    pm, sv = _k2b(md_flat, bin_logit_baseline, genes_pad, cxg_pad, gix_pad,
                  coord_pad)

    pm2 = pm.reshape(NCPAD // 128, 128)
    s2 = sv.reshape(NCPAD // 128, 128)
    h0 = hist[0, :BG].reshape(B, G)
    h1 = hist[1, :BG].reshape(B, G)
    rb_row = rboi[:G].reshape(1, G)
    ls_col = lsoi.reshape(B, 1)

    out = _k3(pm2, s2, h0, h1, latent, rwoi[:G, :L], rb_row, ls_col)
    scale = (jnp.asarray(n_cells, jnp.float32) * jnp.asarray(n_genes, jnp.float32)
             / jnp.float32(BG))
    return out[0, 0] * scale


# speedup vs baseline: 7.0949x; 1.0101x over previous
import functools

import jax
import jax.numpy as jnp
from jax import lax
from jax.experimental import pallas as pl
from jax.experimental.pallas import tpu as pltpu
from jax.experimental.pallas import tpu_sc as plsc

B = 256
G = 500
L = 32
C = 128
NGT = 5000
NC = 200000
NF = 400000
NW = 32
CH = 128
NC_CHUNKS = 1568
NCPAD = NC_CHUNKS * CH
NF_CHUNKS = 3136
NFPAD = NF_CHUNKS * CH
H = 131072
BG = B * G


GPS = 4


def _k1_body(genes_ref, latent_ref, lw_hbm, md_ref, lw_scr, sem):
    j = pl.program_id(0)
    nsteps = pl.num_programs(0)
    latent = latent_ref[...]

    def fire(jj, slot):
        for k in range(GPS):
            pltpu.make_async_copy(lw_hbm.at[genes_ref[jj * GPS + k]],
                                  lw_scr.at[slot, k], sem).start()

    @pl.when(j == 0)
    def _prime():
        fire(0, 0)

    @pl.when(j < nsteps - 1)
    def _next():
        fire(j + 1, lax.rem(j + 1, 2))

    slot = lax.rem(j, 2)
    for k in range(GPS):
        pltpu.make_async_copy(lw_hbm.at[genes_ref[j * GPS + k]],
                              lw_scr.at[slot, k], sem).wait()
        md_ref[pl.ds(k * B, B), :] = jnp.dot(
            latent, lw_scr[slot, k], preferred_element_type=jnp.float32)


def _k1(latent, genes_oi, logit_weight):
    grid_spec = pltpu.PrefetchScalarGridSpec(
        num_scalar_prefetch=1,
        grid=(G // GPS,),
        in_specs=[
            pl.BlockSpec((B, L), lambda g, gref: (0, 0)),
            pl.BlockSpec(memory_space=pl.ANY),
        ],
        out_specs=[
            pl.BlockSpec((GPS * B, C), lambda g, gref: (g, 0)),
        ],
        scratch_shapes=[
            pltpu.VMEM((2, GPS, L, C), jnp.float32),
            pltpu.SemaphoreType.DMA,
        ],
    )
    return pl.pallas_call(
        _k1_body,
        grid_spec=grid_spec,
        out_shape=[
            jax.ShapeDtypeStruct((G * B, C), jnp.float32),
        ],
    )(genes_oi, latent, logit_weight)[0]


def _k2a_body(genes_hbm, frag_hbm, rb_hbm, ls_hbm, cells_hbm, rw_hbm,
              hist_hbm, rboi_hbm, lsoi_hbm, rwoi_hbm,
              genes_v, frag_a, frag_b, ones_v, zeros_v,
              rb_v, ls_v, cells_v, rboi_v, lsoi_v, gidx_v, rwoi_v,
              hist_sh, sem_f_a, sem_f_b, sem1):
    c = lax.axis_index("c")
    s = lax.axis_index("s")
    wid = s * 2 + c
    NJF = NF_CHUNKS // NW
    fbase = wid * (NJF * CH)

    pltpu.sync_copy(genes_hbm, genes_v)

    @pl.when(s == 0)
    def _zero_hist():
        def zv(i, carry):
            zeros_v[pl.ds(i * 16, 16)] = jnp.zeros((16,), jnp.int32)
            return carry
        lax.fori_loop(0, 64, zv, 0)

        def zh(k, carry):
            pltpu.sync_copy(zeros_v, hist_sh.at[pl.ds(k * 1024, 1024)])
            return carry
        lax.fori_loop(0, H // 1024, zh, 0)

    for k in range(8):
        ones_v[pl.ds(k * 16, 16)] = jnp.full((16,), 1, jnp.int32)

    plsc.subcore_barrier()

    def ffire(j, fb, semf):
        toff = pl.multiple_of(fbase + j * CH, CH)
        pltpu.async_copy(frag_hbm.at[pl.ds(toff, CH)], fb, semf)

    ffire(0, frag_a, sem_f_a)

    def frag_step(j, carry):
        even = lax.rem(j, 2) == 0

        @pl.when(jnp.logical_and(even, j < NJF - 1))
        def _fb():
            ffire(j + 1, frag_b, sem_f_b)

        @pl.when(jnp.logical_and(jnp.logical_not(even), j < NJF - 1))
        def _fa():
            ffire(j + 1, frag_a, sem_f_a)

        @pl.when(even)
        def _sa():
            pltpu.make_async_copy(frag_hbm.at[pl.ds(0, CH)], frag_a,
                                  sem_f_a).wait()
            pltpu.sync_copy(ones_v, hist_sh.at[frag_a], add=True)

        @pl.when(jnp.logical_not(even))
        def _sb():
            pltpu.make_async_copy(frag_hbm.at[pl.ds(0, CH)], frag_b,
                                  sem_f_b).wait()
            pltpu.sync_copy(ones_v, hist_sh.at[frag_b], add=True)
        return carry
    lax.fori_loop(0, NJF, frag_step, 0)

    @pl.when(jnp.logical_and(c == 0, s == 0))
    def _small_gathers():
        pltpu.sync_copy(rb_hbm, rb_v)
        pltpu.sync_copy(ls_hbm, ls_v)
        pltpu.sync_copy(cells_hbm, cells_v)
        for k in range(512 // 16):
            sl = pl.ds(k * 16, 16)
            rboi_v[sl] = plsc.load_gather(rb_v, [genes_v[sl]])
        for k in range(256 // 16):
            sl = pl.ds(k * 16, 16)
            lsoi_v[sl] = plsc.load_gather(ls_v, [cells_v[sl]])
        pltpu.sync_copy(rboi_v, rboi_hbm)
        pltpu.sync_copy(lsoi_v, lsoi_hbm)
        for k in range(512 // CH):
            pltpu.sync_copy(genes_hbm.at[pl.ds(k * CH, CH)], gidx_v)
            pltpu.async_copy(rw_hbm.at[gidx_v], rwoi_v, sem1).wait()
            pltpu.sync_copy(rwoi_v, rwoi_hbm.at[pl.ds(k * CH, CH)])

    plsc.subcore_barrier()

    @pl.when(s == 0)
    def _export_hist():
        pltpu.sync_copy(hist_sh, hist_hbm.at[c])


def _k2a(genes_pad, frag_pad, rb_pad, ls, cells, rw):
    kfn = functools.partial(
        pl.kernel,
        out_type=[
            jax.ShapeDtypeStruct((2, H), jnp.int32),
            jax.ShapeDtypeStruct((512,), jnp.float32),
            jax.ShapeDtypeStruct((256,), jnp.float32),
            jax.ShapeDtypeStruct((512, C), jnp.float32),
        ],
        mesh=plsc.VectorSubcoreMesh(core_axis_name="c", subcore_axis_name="s"),
        compiler_params=pltpu.CompilerParams(needs_layout_passes=False),
        scratch_types=[
            pltpu.VMEM((512,), jnp.int32),
            pltpu.VMEM((CH,), jnp.int32),
            pltpu.VMEM((CH,), jnp.int32),
            pltpu.VMEM((CH,), jnp.int32),
            pltpu.VMEM((1024,), jnp.int32),
            pltpu.VMEM((5120,), jnp.float32),
            pltpu.VMEM((10000,), jnp.float32),
            pltpu.VMEM((256,), jnp.int32),
            pltpu.VMEM((512,), jnp.float32),
            pltpu.VMEM((256,), jnp.float32),
            pltpu.VMEM((CH,), jnp.int32),
            pltpu.VMEM((CH, C), jnp.float32),
            pltpu.VMEM_SHARED((H,), jnp.int32),
            pltpu.SemaphoreType.DMA,
            pltpu.SemaphoreType.DMA,
            pltpu.SemaphoreType.DMA,
        ],
    )
    return kfn(_k2a_body)(genes_pad, frag_pad, rb_pad, ls, cells, rw)


def _k2b_body(md_hbm, bl_hbm, genes_hbm, cxg_hbm, gix_hbm, coord_hbm,
              pm_hbm, s_hbm,
              genes_v, idx_all, gix_all, coord_all, pm_v, s_v,
              idx_a, gg_a, bins_a, md_a, bl_a,
              idx_b, gg_b, bins_b, md_b, bl_b,
              sem_md_a, sem_bl_a, sem_md_b, sem_bl_b):
    c = lax.axis_index("c")
    s = lax.axis_index("s")
    wid = s * 2 + c
    NJ = NC_CHUNKS // NW
    cbase = wid * (NJ * CH)

    pltpu.sync_copy(genes_hbm, genes_v)
    pltpu.sync_copy(cxg_hbm.at[pl.ds(cbase, NJ * CH)], idx_all)
    pltpu.sync_copy(gix_hbm.at[pl.ds(cbase, NJ * CH)], gix_all)
    pltpu.sync_copy(coord_hbm.at[pl.ds(cbase, NJ * CH)], coord_all)

    lane = lax.iota(jnp.int32, 16)
    lane0 = lane == 0

    def fire(j, idxb, ggb, binsb, mdb, blb, sem_md, sem_bl):
        for k in range(CH // 16):
            sl = pl.ds(k * 16, 16)
            gl = pl.ds(j * CH + k * 16, 16)
            ix = idx_all[gl]
            idxb[sl] = lax.rem(ix, jnp.int32(G)) * B + lax.div(ix, jnp.int32(G))
            ggb[sl] = plsc.load_gather(genes_v, [gix_all[gl]])
            b = (coord_all[gl] * jnp.float32(C)).astype(jnp.int32)
            binsb[sl] = jnp.clip(b, 0, C - 1)
        pltpu.async_copy(md_hbm.at[idxb], mdb, sem_md)
        pltpu.async_copy(bl_hbm.at[ggb], blb, sem_bl)

    def compute(j, idxb, ggb, binsb, mdb, blb, sem_md, sem_bl):
        pltpu.make_async_copy(md_hbm.at[idxb], mdb, sem_md).wait()
        pltpu.make_async_copy(bl_hbm.at[ggb], blb, sem_bl).wait()
        toff = pl.multiple_of(cbase + j * CH, CH)
        for k in range(CH // 16):
            sl = pl.ds(k * 16, 16)
            rows = lane + jnp.int32(k * 16)
            cols = binsb[sl]
            pm_v[sl] = (plsc.load_gather(mdb, [rows, cols])
                        + plsc.load_gather(blb, [rows, cols]))

        def cut_body(i2, carry2):
            accs = []
            for u in range(8):
                i = i2 * 8 + u
                acc = jnp.zeros((16,), jnp.float32)
                for k in range(C // 16):
                    sl = pl.ds(k * 16, 16)
                    acc = acc + jnp.exp(mdb[i, sl] + blb[i, sl])
                accs.append(acc)
            for u in range(8):
                i = i2 * 8 + u
                ii = jnp.full((16,), i, jnp.int32)
                plsc.store_scatter(s_v, [ii],
                                   jnp.sum(accs[u]) + jnp.zeros((16,),
                                                                jnp.float32),
                                   mask=lane0)
            return carry2
        lax.fori_loop(0, CH // 8, cut_body, 0)

        pltpu.sync_copy(pm_v, pm_hbm.at[pl.ds(toff, CH)])
        pltpu.sync_copy(s_v, s_hbm.at[pl.ds(toff, CH)])

    bufs_a = (idx_a, gg_a, bins_a, md_a, bl_a, sem_md_a, sem_bl_a)
    bufs_b = (idx_b, gg_b, bins_b, md_b, bl_b, sem_md_b, sem_bl_b)
    fire(0, *bufs_a)

    def cut_step(j, carry):
        even = lax.rem(j, 2) == 0

        @pl.when(jnp.logical_and(even, j < NJ - 1))
        def _fb():
            fire(j + 1, *bufs_b)

        @pl.when(jnp.logical_and(jnp.logical_not(even), j < NJ - 1))
        def _fa():
            fire(j + 1, *bufs_a)

        @pl.when(even)
        def _ca():
            compute(j, *bufs_a)

        @pl.when(jnp.logical_not(even))
        def _cb():
            compute(j, *bufs_b)
        return carry
    lax.fori_loop(0, NJ, cut_step, 0)


def _k2b(md_flat, bl, genes_pad, cxg_pad, gix_pad, coord_pad):
    kfn = functools.partial(
        pl.kernel,
        out_type=[
            jax.ShapeDtypeStruct((NCPAD,), jnp.float32),
            jax.ShapeDtypeStruct((NCPAD,), jnp.float32),
        ],
        mesh=plsc.VectorSubcoreMesh(core_axis_name="c", subcore_axis_name="s"),
        compiler_params=pltpu.CompilerParams(needs_layout_passes=False),
        scratch_types=[
            pltpu.VMEM((512,), jnp.int32),
            pltpu.VMEM((NCPAD // NW,), jnp.int32),
            pltpu.VMEM((NCPAD // NW,), jnp.int32),
            pltpu.VMEM((NCPAD // NW,), jnp.float32),
            pltpu.VMEM((CH,), jnp.float32),
            pltpu.VMEM((CH,), jnp.float32),
            pltpu.VMEM((CH,), jnp.int32),
            pltpu.VMEM((CH,), jnp.int32),
            pltpu.VMEM((CH,), jnp.int32),
            pltpu.VMEM((CH, C), jnp.float32),
            pltpu.VMEM((CH, C), jnp.float32),
            pltpu.VMEM((CH,), jnp.int32),
            pltpu.VMEM((CH,), jnp.int32),
            pltpu.VMEM((CH,), jnp.int32),
            pltpu.VMEM((CH, C), jnp.float32),
            pltpu.VMEM((CH, C), jnp.float32),
            pltpu.SemaphoreType.DMA,
            pltpu.SemaphoreType.DMA,
            pltpu.SemaphoreType.DMA,
            pltpu.SemaphoreType.DMA,
        ],
    )
    return kfn(_k2b_body)(md_flat, bl, genes_pad, cxg_pad, gix_pad, coord_pad)


def _k3_body(pm_ref, s_ref, h0_ref, h1_ref, latent_ref, rw_ref, rb_ref,
             ls_ref, out_ref):
    pm = pm_ref[...]
    sv = s_ref[...]
    r0 = lax.broadcasted_iota(jnp.int32, pm.shape, 0)
    c0 = lax.broadcasted_iota(jnp.int32, pm.shape, 1)
    maskc = (r0 * 128 + c0) < NC
    mix = jnp.sum(jnp.where(maskc, pm - jnp.log(jnp.where(maskc, sv, 1.0)),
                            0.0))
    mix = mix + jnp.float32(NC) * jnp.log(jnp.float32(C))

    fc = (h0_ref[...] + h1_ref[...]).astype(jnp.float32)
    rho = lax.dot_general(latent_ref[...], rw_ref[...],
                          (((1,), (1,)), ((), ())),
                          preferred_element_type=jnp.float32,
                          precision=lax.Precision.HIGHEST)
    fe = rb_ref[...] * jnp.exp(rho) * ls_ref[...]
    x = fc + 1.0
    z = x + 7.0
    prod = (x * (x + 1.0) * (x + 2.0) * (x + 3.0) * (x + 4.0) * (x + 5.0)
            * (x + 6.0))
    zi = 1.0 / z
    zi2 = zi * zi
    lg = ((z - 0.5) * jnp.log(z) - z + jnp.float32(0.9189385332046727)
          + zi * (jnp.float32(1.0 / 12.0)
                  - zi2 * (jnp.float32(1.0 / 360.0)
                           - zi2 * jnp.float32(1.0 / 1260.0)))
          - jnp.log(prod))
    lfc = fc * jnp.log(fe) - fe - lg
    out_ref[0, 0] = -(mix + jnp.sum(lfc))


def _k3(pm2, s2, h0, h1, latent, rwoi, rb_row, ls_col):
    return pl.pallas_call(
        _k3_body,
        out_shape=jax.ShapeDtypeStruct((1, 1), jnp.float32),
        out_specs=pl.BlockSpec(memory_space=pltpu.SMEM),
    )(pm2, s2, h0, h1, latent, rwoi, rb_row, ls_col)


def kernel(latent, genes_oi, cells_oi, cut_coordinates, cut_local_cellxgene_ix,
           cut_local_gene_ix, local_cellxgene_ix, n_cells, n_genes,
           logit_weight, rho_weight, bin_logit_baseline, rho_bias, libsize):
    genes_oi = genes_oi.astype(jnp.int32)
    cells_oi = cells_oi.astype(jnp.int32)
    cxg = cut_local_cellxgene_ix.astype(jnp.int32)
    gix = cut_local_gene_ix.astype(jnp.int32)
    frag = local_cellxgene_ix.astype(jnp.int32)

    md_flat = _k1(latent, genes_oi, logit_weight)

    genes_pad = jnp.pad(genes_oi, (0, 512 - G))
    cxg_pad = jnp.pad(cxg, (0, NCPAD - NC))
    gix_pad = jnp.pad(gix, (0, NCPAD - NC))
    coord_pad = jnp.pad(cut_coordinates, (0, NCPAD - NC))
    frag_pad = jnp.pad(frag, (0, NFPAD - NF), constant_values=BG)
    rb_pad = jnp.pad(rho_bias, (0, 5120 - NGT))
    rw_pad = jnp.pad(rho_weight, ((0, 0), (0, C - L)))

    hist, rboi, lsoi, rwoi = _k2a(genes_pad, frag_pad, rb_pad, libsize,
                                  cells_oi, rw_pad)
    pm, sv = _k2b(md_flat, bin_logit_baseline, genes_pad, cxg_pad, gix_pad,
                  coord_pad)

    pm2 = pm.reshape(NCPAD // 128, 128)
    s2 = sv.reshape(NCPAD // 128, 128)
    h0 = hist[0, :BG].reshape(B, G)
    h1 = hist[1, :BG].reshape(B, G)
    rb_row = rboi[:G].reshape(1, G)
    ls_col = lsoi.reshape(B, 1)

    out = _k3(pm2, s2, h0, h1, latent, rwoi[:G, :L], rb_row, ls_col)
    scale = (jnp.asarray(n_cells, jnp.float32) * jnp.asarray(n_genes, jnp.float32)
             / jnp.float32(BG))
    return out[0, 0] * scale
